# Initial kernel scaffold; baseline (speedup 1.0000x reference)
import functools

import jax
import jax.numpy as jnp
from jax import lax
from jax.experimental import pallas as pl
from jax.experimental.pallas import tpu as pltpu
from jax.experimental.pallas import tpu_sc as plsc

N = 10000
NP = 10240
D = 128
B = 128
T = 10
CP = 32
CH = 128
EPAD = 344064
NCHUNK = EPAD // CH
TILE_CHUNKS = NCHUNK // 16
W_CHUNKS = TILE_CHUNKS // 2
STRIPE = NP // 16

_f32 = jnp.float32
_i32 = jnp.int32

_mesh = plsc.VectorSubcoreMesh(core_axis_name="c", subcore_axis_name="s")


def _leaky(u):
    return jnp.where(u >= 0, u, 0.2 * u)


@functools.partial(
    pl.kernel,
    out_type=jax.ShapeDtypeStruct((2 * NP * CP,), _f32),
    mesh=_mesh,
    scratch_types=[
        pltpu.VMEM((NP,), _i32),
        pltpu.VMEM((CP,), _f32),
        pltpu.VMEM((CP,), _f32),
        pltpu.VMEM((TILE_CHUNKS, CH), _i32),
        pltpu.VMEM((TILE_CHUNKS, CH), _i32),
        pltpu.VMEM((TILE_CHUNKS, CH), _f32),
        pltpu.VMEM((TILE_CHUNKS, CH), _i32),
        pltpu.VMEM((NP,), _f32),
        pltpu.VMEM((2048,), _f32),
        pltpu.VMEM((1, CH), _f32),
        pltpu.VMEM((1, CH), _i32),
        pltpu.VMEM_SHARED((NP,), _f32),
        pltpu.VMEM_SHARED((NP * CP,), _f32),
    ],
)
def _sc_layer0(xh, srch, dsth, s0h, d0h, out_h,
               x_v, s0_v, d0_v, src2, dst2, ex2, xs2, den_v, zero_v,
               cf_v, ix_v, sh_den, sh_c):
    s = lax.axis_index("s")
    c = lax.axis_index("c")

    pltpu.sync_copy(xh, x_v)
    pltpu.sync_copy(s0h, s0_v)
    pltpu.sync_copy(d0h, d0_v)
    pltpu.sync_copy(srch.at[pl.ds(s * TILE_CHUNKS, TILE_CHUNKS)], src2)
    pltpu.sync_copy(dsth.at[pl.ds(s * TILE_CHUNKS, TILE_CHUNKS)], dst2)

    ms = jnp.max(jnp.maximum(s0_v[pl.ds(0, 16)], s0_v[pl.ds(16, 16)]))
    md = jnp.max(jnp.maximum(d0_v[pl.ds(0, 16)], d0_v[pl.ds(16, 16)]))
    shift = _leaky(ms + md)

    def _z(i, carry):
        zero_v[pl.ds(i * 16, 16)] = jnp.zeros((16,), _f32)
        return carry
    lax.fori_loop(0, 128, _z, 0)
    pltpu.sync_copy(zero_v.at[pl.ds(0, STRIPE)],
                    sh_den.at[pl.ds(s * STRIPE, STRIPE)])

    def _zc(t, carry):
        pltpu.sync_copy(zero_v, sh_c.at[pl.ds(s * STRIPE * CP + t * 2048, 2048)])
        return carry
    lax.fori_loop(0, STRIPE * CP // 2048, _zc, 0)
    plsc.subcore_barrier()

    def _den(k, carry):
        for j in range(8):
            sl = pl.ds(j * 16, 16)
            s16 = src2[k, sl]
            d16 = dst2[k, sl]
            xs = plsc.load_gather(x_v, [s16])
            xd = plsc.load_gather(x_v, [d16])
            a = plsc.load_gather(s0_v, [xs])
            b = plsc.load_gather(d0_v, [xd])
            ex = jnp.exp(_leaky(a + b) - shift)
            ex2[k, sl] = ex
            xs2[k, sl] = xs
        pltpu.sync_copy(ex2.at[k], sh_den.at[dst2.at[k]], add=True)
        return carry
    lax.fori_loop(0, TILE_CHUNKS, _den, 0)
    plsc.subcore_barrier()

    pltpu.sync_copy(sh_den, den_v)

    def _hist(k2, carry):
        k = c * W_CHUNKS + k2
        for j in range(8):
            sl = pl.ds(j * 16, 16)
            d16 = dst2[k, sl]
            den16 = plsc.load_gather(den_v, [d16])
            coef = ex2[k, sl] / (den16 + 1e-16)
            cf_v[0, sl] = coef
            ix_v[0, sl] = d16 * CP + xs2[k, sl]
        pltpu.sync_copy(cf_v.at[0], sh_c.at[ix_v.at[0]], add=True)
        return carry
    lax.fori_loop(0, W_CHUNKS, _hist, 0)
    plsc.subcore_barrier()

    pltpu.sync_copy(
        sh_c.at[pl.ds(s * STRIPE * CP, STRIPE * CP)],
        out_h.at[pl.ds(c * NP * CP + s * STRIPE * CP, STRIPE * CP)])


@functools.partial(
    pl.kernel,
    out_type=jax.ShapeDtypeStruct((2 * NP, D), _f32),
    mesh=_mesh,
    scratch_types=[
        pltpu.VMEM((NP,), _f32),
        pltpu.VMEM((NP,), _f32),
        pltpu.VMEM((NP,), _f32),
        pltpu.VMEM((TILE_CHUNKS, CH), _i32),
        pltpu.VMEM((TILE_CHUNKS, CH), _i32),
        pltpu.VMEM((TILE_CHUNKS, CH), _f32),
        pltpu.VMEM((CH, D), _f32),
        pltpu.VMEM((32, D), _f32),
        pltpu.VMEM((STRIPE,), _f32),
        pltpu.VMEM_SHARED((NP,), _f32),
        pltpu.VMEM_SHARED((NP, D), _f32),
        pltpu.SemaphoreType.DMA,
    ],
)
def _sc_layer1(hh, ash, adh, srch, dsth, out_h,
               as_v, ad_v, den_v, src2, dst2, ex2, rows_v, zrow_v, zden_v,
               sh_den, sh_out, sem):
    s = lax.axis_index("s")
    c = lax.axis_index("c")

    pltpu.sync_copy(ash, as_v)
    pltpu.sync_copy(adh, ad_v)
    pltpu.sync_copy(srch.at[pl.ds(s * TILE_CHUNKS, TILE_CHUNKS)], src2)
    pltpu.sync_copy(dsth.at[pl.ds(s * TILE_CHUNKS, TILE_CHUNKS)], dst2)

    def _mx(ref):
        def step(i, m):
            return jnp.maximum(m, ref[pl.ds(i * 16, 16)])
        return jnp.max(lax.fori_loop(0, N // 16, step,
                                     jnp.full((16,), -1e30, _f32)))
    shift = _leaky(_mx(as_v) + _mx(ad_v))

    def _z1(i, carry):
        zden_v[pl.ds(i * 16, 16)] = jnp.zeros((16,), _f32)
        return carry
    lax.fori_loop(0, STRIPE // 16, _z1, 0)

    def _z2(i, carry):
        for j in range(8):
            zrow_v[i, pl.ds(j * 16, 16)] = jnp.zeros((16,), _f32)
        return carry
    lax.fori_loop(0, 32, _z2, 0)
    pltpu.sync_copy(zden_v, sh_den.at[pl.ds(s * STRIPE, STRIPE)])

    def _z3(t, carry):
        pltpu.sync_copy(zrow_v, sh_out.at[pl.ds(s * STRIPE + t * 32, 32)])
        return carry
    lax.fori_loop(0, STRIPE // 32, _z3, 0)
    plsc.subcore_barrier()

    def _den(k, carry):
        for j in range(8):
            sl = pl.ds(j * 16, 16)
            a = plsc.load_gather(as_v, [src2[k, sl]])
            b = plsc.load_gather(ad_v, [dst2[k, sl]])
            ex2[k, sl] = jnp.exp(_leaky(a + b) - shift)
        pltpu.sync_copy(ex2.at[k], sh_den.at[dst2.at[k]], add=True)
        return carry
    lax.fori_loop(0, TILE_CHUNKS, _den, 0)
    plsc.subcore_barrier()

    pltpu.sync_copy(sh_den, den_v)

    def _rows(k2, carry):
        k = c * W_CHUNKS + k2
        cp = pltpu.async_copy(hh.at[src2.at[k]], rows_v, sem)
        for j in range(8):
            sl = pl.ds(j * 16, 16)
            den16 = plsc.load_gather(den_v, [dst2[k, sl]])
            ex2[k, sl] = ex2[k, sl] / (den16 + 1e-16)
        cp.wait()

        def _scale(e, carry2):
            cc = ex2[k, e]
            for j in range(8):
                sl = pl.ds(j * 16, 16)
                rows_v[e, sl] = rows_v[e, sl] * cc
            return carry2
        lax.fori_loop(0, CH, _scale, 0)
        pltpu.sync_copy(rows_v, sh_out.at[dst2.at[k]], add=True)
        return carry
    lax.fori_loop(0, W_CHUNKS, _rows, 0)
    plsc.subcore_barrier()

    pltpu.sync_copy(sh_out.at[pl.ds(s * STRIPE, STRIPE)],
                    out_h.at[pl.ds(c * NP + s * STRIPE, STRIPE)])


def _tc_prep(ftp, W0, asw, adw):
    def body(ft_r, w_r, as_r, ad_r, t0_r, s_r, d_r):
        t0 = jnp.dot(ft_r[...], w_r[...], preferred_element_type=_f32)
        t0_r[...] = t0
        rid = lax.broadcasted_iota(_i32, (CP, 1), 0)
        sval = jnp.dot(t0, as_r[...], preferred_element_type=_f32)
        dval = jnp.dot(t0, ad_r[...], preferred_element_type=_f32)
        s_r[...] = jnp.where(rid < 21, sval, -1e30)
        d_r[...] = jnp.where(rid < 21, dval, -1e30)

    return pl.pallas_call(
        body,
        out_shape=[
            jax.ShapeDtypeStruct((CP, D), _f32),
            jax.ShapeDtypeStruct((CP, 1), _f32),
            jax.ShapeDtypeStruct((CP, 1), _f32),
        ],
    )(ftp, W0, asw, adw)


_R = 1024


def _tc_mid(c0, c1, T0p, b0, W1, asw, adw):
    def body(c0_r, c1_r, t0_r, b0_r, w1_r, as_r, ad_r, h_r, s_r, d_r):
        cb = c0_r[...] + c1_r[...]
        h1 = jnp.maximum(
            jnp.dot(cb, t0_r[...], preferred_element_type=_f32) + b0_r[...], 0.0)
        H1 = jnp.dot(h1, w1_r[...], preferred_element_type=_f32)
        h_r[...] = H1
        s_r[...] = jnp.dot(H1, as_r[...], preferred_element_type=_f32)
        d_r[...] = jnp.dot(H1, ad_r[...], preferred_element_type=_f32)

    grid = (NP // _R,)
    return pl.pallas_call(
        body,
        grid=grid,
        in_specs=[
            pl.BlockSpec((_R, CP), lambda i: (i, 0)),
            pl.BlockSpec((_R, CP), lambda i: (i, 0)),
            pl.BlockSpec((CP, D), lambda i: (0, 0)),
            pl.BlockSpec((1, D), lambda i: (0, 0)),
            pl.BlockSpec((D, D), lambda i: (0, 0)),
            pl.BlockSpec((D, 1), lambda i: (0, 0)),
            pl.BlockSpec((D, 1), lambda i: (0, 0)),
        ],
        out_specs=[
            pl.BlockSpec((_R, D), lambda i: (i, 0)),
            pl.BlockSpec((_R, 1), lambda i: (i, 0)),
            pl.BlockSpec((_R, 1), lambda i: (i, 0)),
        ],
        out_shape=[
            jax.ShapeDtypeStruct((NP, D), _f32),
            jax.ShapeDtypeStruct((NP, 1), _f32),
            jax.ShapeDtypeStruct((NP, 1), _f32),
        ],
    )(c0, c1, T0p, b0, W1, asw, adw)


def _tc_final(o0, o1, b1, batchp, fW1, fb1, fW2, fb2):
    def body(o0_r, o1_r, b1_r, bt_r, w1_r, q1_r, w2_r, q2_r, out_r, g_acc):
        i = pl.program_id(0)

        @pl.when(i == 0)
        def _():
            g_acc[...] = jnp.zeros_like(g_acc)

        h2 = jnp.maximum(o0_r[...] + o1_r[...] + b1_r[...], 0.0)
        col = lax.broadcasted_iota(_i32, (_R, B), 1)
        oh = (bt_r[...] == col).astype(_f32)
        g_acc[...] += lax.dot_general(
            oh, h2, (((0,), (0,)), ((), ())), preferred_element_type=_f32)

        @pl.when(i == NP // _R - 1)
        def _():
            g = g_acc[...]
            z = jnp.maximum(
                jnp.dot(g, w1_r[...], preferred_element_type=_f32) + q1_r[...],
                0.0)
            out_r[...] = (jnp.dot(z, w2_r[...], preferred_element_type=_f32)
                          + q2_r[...])

    grid = (NP // _R,)
    return pl.pallas_call(
        body,
        grid=grid,
        in_specs=[
            pl.BlockSpec((_R, D), lambda i: (i, 0)),
            pl.BlockSpec((_R, D), lambda i: (i, 0)),
            pl.BlockSpec((1, D), lambda i: (0, 0)),
            pl.BlockSpec((_R, 1), lambda i: (i, 0)),
            pl.BlockSpec((D, 2 * D), lambda i: (0, 0)),
            pl.BlockSpec((1, 2 * D), lambda i: (0, 0)),
            pl.BlockSpec((2 * D, T), lambda i: (0, 0)),
            pl.BlockSpec((1, T), lambda i: (0, 0)),
        ],
        out_specs=pl.BlockSpec((B, T), lambda i: (0, 0)),
        out_shape=jax.ShapeDtypeStruct((B, T), _f32),
        scratch_shapes=[pltpu.VMEM((B, B), _f32)],
    )(o0, o1, b1, batchp, fW1, fb1, fW2, fb2)


def kernel(x, edge_index, edge_attr, batch, feat_table, W0, att_s0, att_d0, b0,
           W1, att_s1, att_d1, b1, fW1, fb1, fW2, fb2):
    n = x.shape[0]
    e = edge_index.shape[1]

    loop = jnp.arange(n, dtype=_i32)
    npad = EPAD - e - n
    src = jnp.concatenate([edge_index[0].astype(_i32), loop,
                           jnp.zeros((npad,), _i32)]).reshape(NCHUNK, CH)
    dst = jnp.concatenate([edge_index[1].astype(_i32), loop,
                           jnp.full((npad,), n, _i32)]).reshape(NCHUNK, CH)
    xp = jnp.pad(x.astype(_i32), (0, NP - n))
    ftp = jnp.pad(feat_table, ((0, CP - 21), (0, 0)))

    T0p, s0t, d0t = _tc_prep(ftp, W0, att_s0.reshape(D, 1),
                             att_d0.reshape(D, 1))

    cflat = _sc_layer0(xp, src, dst, s0t.reshape(CP), d0t.reshape(CP))
    c2 = cflat.reshape(2, NP, CP)

    H1, as1, ad1 = _tc_mid(c2[0], c2[1], T0p, b0.reshape(1, D), W1,
                           att_s1.reshape(D, 1), att_d1.reshape(D, 1))

    opart = _sc_layer1(H1, as1.reshape(NP), ad1.reshape(NP), src, dst)
    o2 = opart.reshape(2, NP, D)

    batchp = jnp.pad(batch.astype(_i32), (0, NP - n),
                     constant_values=B).reshape(NP, 1)
    return _tc_final(o2[0], o2[1], b1.reshape(1, D), batchp, fW1,
                     fb1.reshape(1, 2 * D), fW2, fb2.reshape(1, T))


# speedup vs baseline: 13.0534x; 13.0534x over previous
import functools

import jax
import jax.numpy as jnp
from jax import lax
from jax.experimental import pallas as pl
from jax.experimental.pallas import tpu as pltpu
from jax.experimental.pallas import tpu_sc as plsc

N = 10000
NP = 10240
HALF = NP // 2
HPAD = 5248
D = 128
B = 128
T = 10
CP = 32
CH = 128
EPAD = 344064
ETILE = EPAD // 16
ECAP = ETILE + CH
HSTRIPE = HPAD // 16

_f32 = jnp.float32
_i32 = jnp.int32


def _leaky(u):
    return jnp.where(u >= 0, u, 0.2 * u)


_GDN = lax.GatherDimensionNumbers(
    offset_dims=(), collapsed_slice_dims=(0,), start_index_map=(0,))


def _perm(v, idx):
    return lax.gather(v, idx[:, None], _GDN, (1,),
                      mode=lax.GatherScatterMode.PROMISE_IN_BOUNDS)


def _lane_max(v):
    idx = lax.iota(_i32, 16)
    for s2 in (8, 4, 2, 1):
        v = jnp.maximum(v, _perm(v, idx ^ s2))
    return v


def _sc_mesh():
    return plsc.VectorSubcoreMesh(core_axis_name="c", subcore_axis_name="s",
                                  num_cores=2, num_subcores=16)


def _pad_chunk(bufs_vals, cnt):
    ii = lax.iota(_i32, 16)
    for j in range(8):
        idx = cnt + j * 16 + ii
        for buf, val in bufs_vals:
            plsc.store_scatter(buf, [idx], jnp.full((16,), val, buf.dtype))


def _chunk_to_ix(buf, kk, ix_v):
    for j in range(8):
        sl = pl.ds(j * 16, 16)
        ix_v[0, sl] = buf[pl.ds(kk * CH + j * 16, 16)]


@functools.lru_cache(maxsize=None)
def _get_sc_layer0():
    @functools.partial(
        pl.kernel,
        out_type=jax.ShapeDtypeStruct((NP * CP,), _f32),
        mesh=_sc_mesh(),
        compiler_params=pltpu.CompilerParams(needs_layout_passes=False),
        scratch_types=[
            pltpu.VMEM((NP,), _i32),
            pltpu.VMEM((CP,), _f32),
            pltpu.VMEM((CP,), _f32),
            pltpu.VMEM((ECAP,), _i32),
            pltpu.VMEM((ECAP,), _i32),
            pltpu.VMEM((ECAP,), _f32),
            pltpu.VMEM((HPAD,), _f32),
            pltpu.VMEM((2624,), _f32),
            pltpu.VMEM((1, CH), _i32),
            pltpu.VMEM((1, CH), _i32),
            pltpu.VMEM_SHARED((HPAD,), _f32),
            pltpu.VMEM_SHARED((HPAD * CP,), _f32),
        ],
    )
    def sc_layer0(xh, srch, dsth, s0h, d0h, out_h,
                  x_v, s0_v, d0_v, src1, dst1, ex1, den_v, zero_v,
                  ix_v, ix2_v, sh_den, sh_c):
        s = lax.axis_index("s")
        c = lax.axis_index("c")
        base = c * HALF

        pltpu.sync_copy(xh, x_v)
        pltpu.sync_copy(s0h, s0_v)
        pltpu.sync_copy(d0h, d0_v)
        pltpu.sync_copy(srch.at[pl.ds(s * ETILE, ETILE)],
                        src1.at[pl.ds(0, ETILE)])
        pltpu.sync_copy(dsth.at[pl.ds(s * ETILE, ETILE)],
                        dst1.at[pl.ds(0, ETILE)])

        ms = _lane_max(jnp.maximum(s0_v[pl.ds(0, 16)], s0_v[pl.ds(16, 16)]))
        md = _lane_max(jnp.maximum(d0_v[pl.ds(0, 16)], d0_v[pl.ds(16, 16)]))
        shift = _leaky(ms + md)

        def _z(i, carry):
            zero_v[pl.ds(i * 16, 16)] = jnp.zeros((16,), _f32)
            return carry
        lax.fori_loop(0, 164, _z, 0)
        pltpu.sync_copy(zero_v.at[pl.ds(0, HSTRIPE)],
                        sh_den.at[pl.ds(s * HSTRIPE, HSTRIPE)])

        def _zc(t, carry):
            pltpu.sync_copy(
                zero_v, sh_c.at[pl.ds((s * 4 + t) * 2624, 2624)])
            return carry
        lax.fori_loop(0, 4, _zc, 0)
        plsc.subcore_barrier()

        def _scan(g, cnt):
            sl = pl.ds(g * 16, 16)
            s16 = src1[sl]
            d16 = dst1[sl]
            xs = plsc.load_gather(x_v, [s16])
            xd = plsc.load_gather(x_v, [d16])
            a = plsc.load_gather(s0_v, [xs])
            b = plsc.load_gather(d0_v, [xd])
            ex = jnp.exp(_leaky(a + b) - shift)
            ld = d16 - base
            m = (ld >= 0) & (ld < HALF)
            plsc.store_compressed(src1.at[pl.ds(cnt, 16)], xs, mask=m)
            plsc.store_compressed(dst1.at[pl.ds(cnt, 16)], ld, mask=m)
            plsc.store_compressed(ex1.at[pl.ds(cnt, 16)], ex, mask=m)
            return cnt + plsc.all_reduce_population_count(m)[0]
        cnt = lax.fori_loop(0, ETILE // 16, _scan, 0)
        _pad_chunk([(src1, 0), (dst1, HALF), (ex1, 0.0)], cnt)
        nk = (cnt + CH - 1) // CH

        def _den(kk, carry):
            _chunk_to_ix(dst1, kk, ix_v)
            pltpu.sync_copy(ex1.at[pl.ds(kk * CH, CH)],
                            sh_den.at[ix_v.at[0]], add=True)
            return carry
        lax.fori_loop(0, nk, _den, 0)
        plsc.subcore_barrier()

        pltpu.sync_copy(sh_den, den_v)

        def _hist(kk, carry):
            for j in range(8):
                sl = pl.ds(kk * CH + j * 16, 16)
                d16 = dst1[sl]
                den16 = plsc.load_gather(den_v, [d16])
                ex1[sl] = ex1[sl] / (den16 + 1e-16)
                ix2_v[0, pl.ds(j * 16, 16)] = d16 * CP + src1[sl]
            pltpu.sync_copy(ex1.at[pl.ds(kk * CH, CH)],
                            sh_c.at[ix2_v.at[0]], add=True)
            return carry
        lax.fori_loop(0, nk, _hist, 0)
        plsc.subcore_barrier()

        pltpu.sync_copy(
            sh_c.at[pl.ds(s * (HALF // 16) * CP, (HALF // 16) * CP)],
            out_h.at[pl.ds(c * HALF * CP + s * (HALF // 16) * CP,
                           (HALF // 16) * CP)])

    return sc_layer0


@functools.lru_cache(maxsize=None)
def _get_sc_layer1():
    @functools.partial(
        pl.kernel,
        out_type=jax.ShapeDtypeStruct((NP, D), _f32),
        mesh=_sc_mesh(),
        compiler_params=pltpu.CompilerParams(needs_layout_passes=False),
        scratch_types=[
            pltpu.VMEM((NP,), _f32),
            pltpu.VMEM((HPAD,), _f32),
            pltpu.VMEM((ECAP,), _i32),
            pltpu.VMEM((ECAP,), _i32),
            pltpu.VMEM((HPAD,), _f32),
            pltpu.VMEM((CH, D), _f32),
            pltpu.VMEM((8, D), _f32),
            pltpu.VMEM((HSTRIPE,), _f32),
            pltpu.VMEM((1, CH), _i32),
            pltpu.VMEM((1, CH), _i32),
            pltpu.VMEM((1, CH), _f32),
            pltpu.VMEM_SHARED((HPAD,), _f32),
            pltpu.VMEM_SHARED((HPAD, D), _f32),
            pltpu.SemaphoreType.DMA,
        ],
    )
    def sc_layer1(hh, ash, adh, srch, dsth, out_h,
                  as_v, adh_v, src1, dst1, den_v, rows_v, zrow_v, zden_v,
                  ix_v, is_v, exc_v, sh_den, sh_out, sem):
        s = lax.axis_index("s")
        c = lax.axis_index("c")
        base = c * HALF

        pltpu.sync_copy(ash, as_v)
        pltpu.sync_copy(adh.at[pl.ds(base, HALF)], adh_v.at[pl.ds(0, HALF)])
        for j in range(8):
            adh_v[pl.ds(HALF + j * 16, 16)] = jnp.zeros((16,), _f32)
        pltpu.sync_copy(srch.at[pl.ds(s * ETILE, ETILE)],
                        src1.at[pl.ds(0, ETILE)])
        pltpu.sync_copy(dsth.at[pl.ds(s * ETILE, ETILE)],
                        dst1.at[pl.ds(0, ETILE)])

        def _mx(ref, ng):
            def step(i, m):
                return jnp.maximum(m, ref[pl.ds(i * 16, 16)])
            return _lane_max(lax.fori_loop(0, ng, step,
                                           jnp.full((16,), -1e30, _f32)))
        ngroups = 305 + (1 - c) * 15
        shift = _leaky(_mx(as_v, N // 16) + _mx(adh_v, ngroups))

        def _z1(i, carry):
            zden_v[pl.ds(i * 16, 16)] = jnp.zeros((16,), _f32)
            return carry
        lax.fori_loop(0, HSTRIPE // 16, _z1, 0)

        def _z2(i, carry):
            for j in range(8):
                zrow_v[i, pl.ds(j * 16, 16)] = jnp.zeros((16,), _f32)
            return carry
        lax.fori_loop(0, 8, _z2, 0)
        pltpu.sync_copy(zden_v, sh_den.at[pl.ds(s * HSTRIPE, HSTRIPE)])

        def _z3(t, carry):
            pltpu.sync_copy(zrow_v,
                            sh_out.at[pl.ds(s * HSTRIPE + t * 8, 8)])
            return carry
        lax.fori_loop(0, HSTRIPE // 8, _z3, 0)
        plsc.subcore_barrier()

        def _scan(g, cnt):
            sl = pl.ds(g * 16, 16)
            s16 = src1[sl]
            d16 = dst1[sl]
            ld = d16 - base
            m = (ld >= 0) & (ld < HALF)
            plsc.store_compressed(src1.at[pl.ds(cnt, 16)], s16, mask=m)
            plsc.store_compressed(dst1.at[pl.ds(cnt, 16)], ld, mask=m)
            return cnt + plsc.all_reduce_population_count(m)[0]
        cnt = lax.fori_loop(0, ETILE // 16, _scan, 0)
        _pad_chunk([(src1, 0), (dst1, HALF)], cnt)
        nk = (cnt + CH - 1) // CH

        def _ex16(kk, j):
            sl = pl.ds(kk * CH + j * 16, 16)
            a = plsc.load_gather(as_v, [src1[sl]])
            b = plsc.load_gather(adh_v, [dst1[sl]])
            return jnp.exp(_leaky(a + b) - shift)

        def _den(kk, carry):
            _chunk_to_ix(dst1, kk, ix_v)
            for j in range(8):
                exc_v[0, pl.ds(j * 16, 16)] = _ex16(kk, j)
            pltpu.sync_copy(exc_v.at[0], sh_den.at[ix_v.at[0]], add=True)
            return carry
        lax.fori_loop(0, nk, _den, 0)
        plsc.subcore_barrier()

        pltpu.sync_copy(sh_den, den_v)

        def _rows(kk, carry):
            _chunk_to_ix(src1, kk, is_v)
            cp = pltpu.async_copy(hh.at[is_v.at[0]], rows_v, sem)
            _chunk_to_ix(dst1, kk, ix_v)
            cp.wait()
            for j in range(8):
                sl = pl.ds(kk * CH + j * 16, 16)
                den16 = plsc.load_gather(den_v, [dst1[sl]])
                coef16 = _ex16(kk, j) / (den16 + 1e-16)
                for l in range(16):
                    cc = coef16[l]
                    e = j * 16 + l
                    for jj in range(8):
                        sl2 = pl.ds(jj * 16, 16)
                        rows_v[e, sl2] = rows_v[e, sl2] * cc
            pltpu.sync_copy(rows_v, sh_out.at[ix_v.at[0]], add=True)
            return carry
        lax.fori_loop(0, nk, _rows, 0)
        plsc.subcore_barrier()

        pltpu.sync_copy(sh_out.at[pl.ds(s * (HALF // 16), HALF // 16)],
                        out_h.at[pl.ds(c * HALF + s * (HALF // 16),
                                       HALF // 16)])

    return sc_layer1


def _tc_prep(ftp, W0, asw, adw):
    def body(ft_r, w_r, as_r, ad_r, t0_r, s_r, d_r):
        t0 = jnp.dot(ft_r[...], w_r[...], preferred_element_type=_f32)
        t0_r[...] = t0
        rid = lax.broadcasted_iota(_i32, (CP, 1), 0)
        sval = jnp.dot(t0, as_r[...], preferred_element_type=_f32)
        dval = jnp.dot(t0, ad_r[...], preferred_element_type=_f32)
        s_r[...] = jnp.where(rid < 21, sval, -1e30)
        d_r[...] = jnp.where(rid < 21, dval, -1e30)

    return pl.pallas_call(
        body,
        out_shape=[
            jax.ShapeDtypeStruct((CP, D), _f32),
            jax.ShapeDtypeStruct((CP, 1), _f32),
            jax.ShapeDtypeStruct((CP, 1), _f32),
        ],
    )(ftp, W0, asw, adw)


_R = 1024


def _tc_mid(cmat, T0p, b0, W1, asw, adw):
    def body(c_r, t0_r, b0_r, w1_r, as_r, ad_r, h_r, s_r, d_r):
        h1 = jnp.maximum(
            jnp.dot(c_r[...], t0_r[...], preferred_element_type=_f32)
            + b0_r[...], 0.0)
        H1 = jnp.dot(h1, w1_r[...], preferred_element_type=_f32)
        h_r[...] = H1
        s_r[...] = jnp.dot(H1, as_r[...], preferred_element_type=_f32)
        d_r[...] = jnp.dot(H1, ad_r[...], preferred_element_type=_f32)

    grid = (NP // _R,)
    return pl.pallas_call(
        body,
        grid=grid,
        in_specs=[
            pl.BlockSpec((_R, CP), lambda i: (i, 0)),
            pl.BlockSpec((CP, D), lambda i: (0, 0)),
            pl.BlockSpec((1, D), lambda i: (0, 0)),
            pl.BlockSpec((D, D), lambda i: (0, 0)),
            pl.BlockSpec((D, 1), lambda i: (0, 0)),
            pl.BlockSpec((D, 1), lambda i: (0, 0)),
        ],
        out_specs=[
            pl.BlockSpec((_R, D), lambda i: (i, 0)),
            pl.BlockSpec((_R, 1), lambda i: (i, 0)),
            pl.BlockSpec((_R, 1), lambda i: (i, 0)),
        ],
        out_shape=[
            jax.ShapeDtypeStruct((NP, D), _f32),
            jax.ShapeDtypeStruct((NP, 1), _f32),
            jax.ShapeDtypeStruct((NP, 1), _f32),
        ],
    )(cmat, T0p, b0, W1, asw, adw)


def _tc_final(o, b1, batchp, fW1, fb1, fW2, fb2):
    def body(o_r, b1_r, bt_r, w1_r, q1_r, w2_r, q2_r, out_r, g_acc):
        i = pl.program_id(0)

        @pl.when(i == 0)
        def _():
            g_acc[...] = jnp.zeros_like(g_acc)

        h2 = jnp.maximum(o_r[...] + b1_r[...], 0.0)
        col = lax.broadcasted_iota(_i32, (_R, B), 1)
        oh = (bt_r[...] == col).astype(_f32)
        g_acc[...] += lax.dot_general(
            oh, h2, (((0,), (0,)), ((), ())), preferred_element_type=_f32)

        @pl.when(i == NP // _R - 1)
        def _():
            g = g_acc[...]
            z = jnp.maximum(
                jnp.dot(g, w1_r[...], preferred_element_type=_f32) + q1_r[...],
                0.0)
            out_r[...] = (jnp.dot(z, w2_r[...], preferred_element_type=_f32)
                          + q2_r[...])

    grid = (NP // _R,)
    return pl.pallas_call(
        body,
        grid=grid,
        in_specs=[
            pl.BlockSpec((_R, D), lambda i: (i, 0)),
            pl.BlockSpec((1, D), lambda i: (0, 0)),
            pl.BlockSpec((_R, 1), lambda i: (i, 0)),
            pl.BlockSpec((D, 2 * D), lambda i: (0, 0)),
            pl.BlockSpec((1, 2 * D), lambda i: (0, 0)),
            pl.BlockSpec((2 * D, T), lambda i: (0, 0)),
            pl.BlockSpec((1, T), lambda i: (0, 0)),
        ],
        out_specs=pl.BlockSpec((B, T), lambda i: (0, 0)),
        out_shape=jax.ShapeDtypeStruct((B, T), _f32),
        scratch_shapes=[pltpu.VMEM((B, B), _f32)],
    )(o, b1, batchp, fW1, fb1, fW2, fb2)


def kernel(x, edge_index, edge_attr, batch, feat_table, W0, att_s0, att_d0, b0,
           W1, att_s1, att_d1, b1, fW1, fb1, fW2, fb2):
    n = x.shape[0]
    e = edge_index.shape[1]

    loop = jnp.arange(n, dtype=_i32)
    npad = EPAD - e - n
    src = jnp.concatenate([edge_index[0].astype(_i32), loop,
                           jnp.zeros((npad,), _i32)])
    dst = jnp.concatenate([edge_index[1].astype(_i32), loop,
                           jnp.full((npad,), n, _i32)])
    xp = jnp.pad(x.astype(_i32), (0, NP - n))
    ftp = jnp.pad(feat_table, ((0, CP - 21), (0, 0)))

    T0p, s0t, d0t = _tc_prep(ftp, W0, att_s0.reshape(D, 1),
                             att_d0.reshape(D, 1))

    cflat = _get_sc_layer0()(xp, src, dst, s0t.reshape(CP), d0t.reshape(CP))
    cmat = cflat.reshape(NP, CP)

    H1, as1, ad1 = _tc_mid(cmat, T0p, b0.reshape(1, D), W1,
                           att_s1.reshape(D, 1), att_d1.reshape(D, 1))

    o1 = _get_sc_layer1()(H1, as1.reshape(NP), ad1.reshape(NP), src, dst)

    batchp = jnp.pad(batch.astype(_i32), (0, NP - n),
                     constant_values=B).reshape(NP, 1)
    return _tc_final(o1, b1.reshape(1, D), batchp, fW1,
                     fb1.reshape(1, 2 * D), fW2, fb2.reshape(1, T))


# speedup vs baseline: 38.5080x; 2.9500x over previous
import functools

import jax
import jax.numpy as jnp
from jax import lax
from jax.experimental import pallas as pl
from jax.experimental.pallas import tpu as pltpu
from jax.experimental.pallas import tpu_sc as plsc

N = 10000
NP = 10240
HALF = NP // 2
HPAD = 5248
D = 128
B = 128
T = 10
CP = 32
CH = 128
EPAD = 344064
ETILE = EPAD // 16
ECAP = ETILE + CH
HSTRIPE = HPAD // 16

_f32 = jnp.float32
_i32 = jnp.int32


def _leaky(u):
    return jnp.where(u >= 0, u, 0.2 * u)


_GDN = lax.GatherDimensionNumbers(
    offset_dims=(), collapsed_slice_dims=(0,), start_index_map=(0,))


def _perm(v, idx):
    return lax.gather(v, idx[:, None], _GDN, (1,),
                      mode=lax.GatherScatterMode.PROMISE_IN_BOUNDS)


def _lane_max(v):
    idx = lax.iota(_i32, 16)
    for s2 in (8, 4, 2, 1):
        v = jnp.maximum(v, _perm(v, idx ^ s2))
    return v


def _sc_mesh():
    return plsc.VectorSubcoreMesh(core_axis_name="c", subcore_axis_name="s",
                                  num_cores=2, num_subcores=16)


def _pad_chunk(src1, dst1, extra, cnt):
    ii = lax.iota(_i32, 16)
    for j in range(8):
        idx = cnt + j * 16 + ii
        plsc.store_scatter(src1, [idx], jnp.zeros((16,), _i32))
        plsc.store_scatter(dst1, [idx], HALF + j * 16 + ii)
        for buf, val in extra:
            plsc.store_scatter(buf, [idx], jnp.full((16,), val, buf.dtype))


def _chunk_to_ix(buf, kk, ix_v):
    for j in range(8):
        sl = pl.ds(j * 16, 16)
        ix_v[0, sl] = buf[pl.ds(kk * CH + j * 16, 16)]


@functools.lru_cache(maxsize=None)
def _get_sc_layer0():
    @functools.partial(
        pl.kernel,
        out_type=jax.ShapeDtypeStruct((NP * CP,), _f32),
        mesh=_sc_mesh(),
        compiler_params=pltpu.CompilerParams(needs_layout_passes=False),
        scratch_types=[
            pltpu.VMEM((NP,), _i32),
            pltpu.VMEM((CP,), _f32),
            pltpu.VMEM((CP,), _f32),
            pltpu.VMEM((ECAP,), _i32),
            pltpu.VMEM((ECAP,), _i32),
            pltpu.VMEM((ECAP,), _f32),
            pltpu.VMEM((HPAD,), _f32),
            pltpu.VMEM((2624,), _f32),
            pltpu.VMEM((1, CH), _i32),
            pltpu.VMEM((1, CH), _i32),
            pltpu.VMEM_SHARED((HPAD,), _f32),
            pltpu.VMEM_SHARED((HPAD * CP,), _f32),
        ],
    )
    def sc_layer0(xh, srch, dsth, s0h, d0h, out_h,
                  x_v, s0_v, d0_v, src1, dst1, ex1, den_v, zero_v,
                  ix_v, ix2_v, sh_den, sh_c):
        s = lax.axis_index("s")
        c = lax.axis_index("c")
        base = c * HALF

        pltpu.sync_copy(xh, x_v)
        pltpu.sync_copy(s0h, s0_v)
        pltpu.sync_copy(d0h, d0_v)
        pltpu.sync_copy(srch.at[pl.ds(s * ETILE, ETILE)],
                        src1.at[pl.ds(0, ETILE)])
        pltpu.sync_copy(dsth.at[pl.ds(s * ETILE, ETILE)],
                        dst1.at[pl.ds(0, ETILE)])

        ms = _lane_max(jnp.maximum(s0_v[pl.ds(0, 16)], s0_v[pl.ds(16, 16)]))
        md = _lane_max(jnp.maximum(d0_v[pl.ds(0, 16)], d0_v[pl.ds(16, 16)]))
        shift = _leaky(ms + md)

        def _z(i, carry):
            zero_v[pl.ds(i * 16, 16)] = jnp.zeros((16,), _f32)
            return carry
        lax.fori_loop(0, 164, _z, 0)
        pltpu.sync_copy(zero_v.at[pl.ds(0, HSTRIPE)],
                        sh_den.at[pl.ds(s * HSTRIPE, HSTRIPE)])

        def _zc(t, carry):
            pltpu.sync_copy(
                zero_v, sh_c.at[pl.ds((s * 4 + t) * 2624, 2624)])
            return carry
        lax.fori_loop(0, 4, _zc, 0)
        plsc.subcore_barrier()

        def _scan(g, cnt):
            sl = pl.ds(g * 16, 16)
            s16 = src1[sl]
            d16 = dst1[sl]
            xs = plsc.load_gather(x_v, [s16])
            xd = plsc.load_gather(x_v, [jnp.minimum(d16, NP - 1)])
            a = plsc.load_gather(s0_v, [xs])
            b = plsc.load_gather(d0_v, [xd])
            ex = jnp.exp(_leaky(a + b) - shift)
            ld = d16 - base
            m = (ld >= 0) & (ld < HALF)
            plsc.store_compressed(src1.at[pl.ds(cnt, 16)], xs, mask=m)
            plsc.store_compressed(dst1.at[pl.ds(cnt, 16)], ld, mask=m)
            plsc.store_compressed(ex1.at[pl.ds(cnt, 16)], ex, mask=m)
            return cnt + plsc.all_reduce_population_count(m)[0]
        cnt = lax.fori_loop(0, ETILE // 16, _scan, 0)
        _pad_chunk(src1, dst1, [(ex1, 0.0)], cnt)
        nk = (cnt + CH - 1) // CH

        def _den(kk, carry):
            _chunk_to_ix(dst1, kk, ix_v)
            pltpu.sync_copy(ex1.at[pl.ds(kk * CH, CH)],
                            sh_den.at[ix_v.at[0]], add=True)
            return carry
        lax.fori_loop(0, nk, _den, 0)
        plsc.subcore_barrier()

        pltpu.sync_copy(sh_den, den_v)

        def _hist(kk, carry):
            for j in range(8):
                sl = pl.ds(kk * CH + j * 16, 16)
                d16 = dst1[sl]
                den16 = plsc.load_gather(den_v, [d16])
                ex1[sl] = ex1[sl] / (den16 + 1e-16)
                ix2_v[0, pl.ds(j * 16, 16)] = d16 * CP + src1[sl]
            pltpu.sync_copy(ex1.at[pl.ds(kk * CH, CH)],
                            sh_c.at[ix2_v.at[0]], add=True)
            return carry
        lax.fori_loop(0, nk, _hist, 0)
        plsc.subcore_barrier()

        pltpu.sync_copy(
            sh_c.at[pl.ds(s * (HALF // 16) * CP, (HALF // 16) * CP)],
            out_h.at[pl.ds(c * HALF * CP + s * (HALF // 16) * CP,
                           (HALF // 16) * CP)])

    return sc_layer0


@functools.lru_cache(maxsize=None)
def _get_sc_layer1():
    @functools.partial(
        pl.kernel,
        out_type=jax.ShapeDtypeStruct((NP, D), _f32),
        mesh=_sc_mesh(),
        compiler_params=pltpu.CompilerParams(needs_layout_passes=False),
        scratch_types=[
            pltpu.VMEM((NP,), _f32),
            pltpu.VMEM((HPAD,), _f32),
            pltpu.VMEM((ECAP,), _i32),
            pltpu.VMEM((ECAP,), _i32),
            pltpu.VMEM((HPAD,), _f32),
            pltpu.VMEM((CH, D), _f32),
            pltpu.VMEM((8, D), _f32),
            pltpu.VMEM((HSTRIPE,), _f32),
            pltpu.VMEM((1, CH), _i32),
            pltpu.VMEM((1, CH), _i32),
            pltpu.VMEM((1, CH), _f32),
            pltpu.VMEM_SHARED((HPAD,), _f32),
            pltpu.VMEM_SHARED((HPAD, D), _f32),
            pltpu.SemaphoreType.DMA,
        ],
    )
    def sc_layer1(hh, ash, adh, srch, dsth, out_h,
                  as_v, adh_v, src1, dst1, den_v, rows_v, zrow_v, zden_v,
                  ix_v, is_v, exc_v, sh_den, sh_out, sem):
        s = lax.axis_index("s")
        c = lax.axis_index("c")
        base = c * HALF

        pltpu.sync_copy(ash, as_v)
        pltpu.sync_copy(adh.at[pl.ds(base, HALF)], adh_v.at[pl.ds(0, HALF)])
        for j in range(8):
            adh_v[pl.ds(HALF + j * 16, 16)] = jnp.zeros((16,), _f32)
        pltpu.sync_copy(srch.at[pl.ds(s * ETILE, ETILE)],
                        src1.at[pl.ds(0, ETILE)])
        pltpu.sync_copy(dsth.at[pl.ds(s * ETILE, ETILE)],
                        dst1.at[pl.ds(0, ETILE)])

        def _mx(ref, ng):
            def step(i, m):
                return jnp.maximum(m, ref[pl.ds(i * 16, 16)])
            return _lane_max(lax.fori_loop(0, ng, step,
                                           jnp.full((16,), -1e30, _f32)))
        ngroups = 305 + (1 - c) * 15
        shift = _leaky(_mx(as_v, N // 16) + _mx(adh_v, ngroups))

        def _z1(i, carry):
            zden_v[pl.ds(i * 16, 16)] = jnp.zeros((16,), _f32)
            return carry
        lax.fori_loop(0, HSTRIPE // 16, _z1, 0)

        def _z2(i, carry):
            for j in range(8):
                zrow_v[i, pl.ds(j * 16, 16)] = jnp.zeros((16,), _f32)
            return carry
        lax.fori_loop(0, 8, _z2, 0)
        pltpu.sync_copy(zden_v, sh_den.at[pl.ds(s * HSTRIPE, HSTRIPE)])

        def _z3(t, carry):
            pltpu.sync_copy(zrow_v,
                            sh_out.at[pl.ds(s * HSTRIPE + t * 8, 8)])
            return carry
        lax.fori_loop(0, HSTRIPE // 8, _z3, 0)
        plsc.subcore_barrier()

        def _scan(g, cnt):
            sl = pl.ds(g * 16, 16)
            s16 = src1[sl]
            d16 = dst1[sl]
            ld = d16 - base
            m = (ld >= 0) & (ld < HALF)
            plsc.store_compressed(src1.at[pl.ds(cnt, 16)], s16, mask=m)
            plsc.store_compressed(dst1.at[pl.ds(cnt, 16)], ld, mask=m)
            return cnt + plsc.all_reduce_population_count(m)[0]
        cnt = lax.fori_loop(0, ETILE // 16, _scan, 0)
        _pad_chunk(src1, dst1, [], cnt)
        nk = (cnt + CH - 1) // CH

        def _ex16(kk, j):
            sl = pl.ds(kk * CH + j * 16, 16)
            a = plsc.load_gather(as_v, [src1[sl]])
            b = plsc.load_gather(adh_v, [dst1[sl]])
            return jnp.exp(_leaky(a + b) - shift)

        def _den(kk, carry):
            _chunk_to_ix(dst1, kk, ix_v)
            for j in range(8):
                exc_v[0, pl.ds(j * 16, 16)] = _ex16(kk, j)
            pltpu.sync_copy(exc_v.at[0], sh_den.at[ix_v.at[0]], add=True)
            return carry
        lax.fori_loop(0, nk, _den, 0)
        plsc.subcore_barrier()

        pltpu.sync_copy(sh_den, den_v)

        def _rows(kk, carry):
            _chunk_to_ix(src1, kk, is_v)
            cp = pltpu.async_copy(hh.at[is_v.at[0]], rows_v, sem)
            _chunk_to_ix(dst1, kk, ix_v)
            cp.wait()
            for j in range(8):
                sl = pl.ds(kk * CH + j * 16, 16)
                den16 = plsc.load_gather(den_v, [dst1[sl]])
                coef16 = _ex16(kk, j) / (den16 + 1e-16)
                for l in range(16):
                    cc = coef16[l]
                    e = j * 16 + l
                    for jj in range(8):
                        sl2 = pl.ds(jj * 16, 16)
                        rows_v[e, sl2] = rows_v[e, sl2] * cc
            pltpu.sync_copy(rows_v, sh_out.at[ix_v.at[0]], add=True)
            return carry
        lax.fori_loop(0, nk, _rows, 0)
        plsc.subcore_barrier()

        pltpu.sync_copy(sh_out.at[pl.ds(s * (HALF // 16), HALF // 16)],
                        out_h.at[pl.ds(c * HALF + s * (HALF // 16),
                                       HALF // 16)])

    return sc_layer1


def _tc_prep(ftp, W0, asw, adw):
    def body(ft_r, w_r, as_r, ad_r, t0_r, s_r, d_r):
        t0 = jnp.dot(ft_r[...], w_r[...], preferred_element_type=_f32)
        t0_r[...] = t0
        rid = lax.broadcasted_iota(_i32, (CP, 1), 0)
        sval = jnp.dot(t0, as_r[...], preferred_element_type=_f32)
        dval = jnp.dot(t0, ad_r[...], preferred_element_type=_f32)
        s_r[...] = jnp.where(rid < 21, sval, -1e30)
        d_r[...] = jnp.where(rid < 21, dval, -1e30)

    return pl.pallas_call(
        body,
        out_shape=[
            jax.ShapeDtypeStruct((CP, D), _f32),
            jax.ShapeDtypeStruct((CP, 1), _f32),
            jax.ShapeDtypeStruct((CP, 1), _f32),
        ],
    )(ftp, W0, asw, adw)


_R = 1024


def _tc_mid(cmat, T0p, b0, W1, asw, adw):
    def body(c_r, t0_r, b0_r, w1_r, as_r, ad_r, h_r, s_r, d_r):
        h1 = jnp.maximum(
            jnp.dot(c_r[...], t0_r[...], preferred_element_type=_f32)
            + b0_r[...], 0.0)
        H1 = jnp.dot(h1, w1_r[...], preferred_element_type=_f32)
        h_r[...] = H1
        s_r[...] = jnp.dot(H1, as_r[...], preferred_element_type=_f32)
        d_r[...] = jnp.dot(H1, ad_r[...], preferred_element_type=_f32)

    grid = (NP // _R,)
    return pl.pallas_call(
        body,
        grid=grid,
        in_specs=[
            pl.BlockSpec((_R, CP), lambda i: (i, 0)),
            pl.BlockSpec((CP, D), lambda i: (0, 0)),
            pl.BlockSpec((1, D), lambda i: (0, 0)),
            pl.BlockSpec((D, D), lambda i: (0, 0)),
            pl.BlockSpec((D, 1), lambda i: (0, 0)),
            pl.BlockSpec((D, 1), lambda i: (0, 0)),
        ],
        out_specs=[
            pl.BlockSpec((_R, D), lambda i: (i, 0)),
            pl.BlockSpec((_R, 1), lambda i: (i, 0)),
            pl.BlockSpec((_R, 1), lambda i: (i, 0)),
        ],
        out_shape=[
            jax.ShapeDtypeStruct((NP, D), _f32),
            jax.ShapeDtypeStruct((NP, 1), _f32),
            jax.ShapeDtypeStruct((NP, 1), _f32),
        ],
    )(cmat, T0p, b0, W1, asw, adw)


def _tc_final(o, b1, batchp, fW1, fb1, fW2, fb2):
    def body(o_r, b1_r, bt_r, w1_r, q1_r, w2_r, q2_r, out_r, g_acc):
        i = pl.program_id(0)

        @pl.when(i == 0)
        def _():
            g_acc[...] = jnp.zeros_like(g_acc)

        h2 = jnp.maximum(o_r[...] + b1_r[...], 0.0)
        col = lax.broadcasted_iota(_i32, (_R, B), 1)
        oh = (bt_r[...] == col).astype(_f32)
        g_acc[...] += lax.dot_general(
            oh, h2, (((0,), (0,)), ((), ())), preferred_element_type=_f32)

        @pl.when(i == NP // _R - 1)
        def _():
            g = g_acc[...]
            z = jnp.maximum(
                jnp.dot(g, w1_r[...], preferred_element_type=_f32) + q1_r[...],
                0.0)
            out_r[...] = (jnp.dot(z, w2_r[...], preferred_element_type=_f32)
                          + q2_r[...])

    grid = (NP // _R,)
    return pl.pallas_call(
        body,
        grid=grid,
        in_specs=[
            pl.BlockSpec((_R, D), lambda i: (i, 0)),
            pl.BlockSpec((1, D), lambda i: (0, 0)),
            pl.BlockSpec((_R, 1), lambda i: (i, 0)),
            pl.BlockSpec((D, 2 * D), lambda i: (0, 0)),
            pl.BlockSpec((1, 2 * D), lambda i: (0, 0)),
            pl.BlockSpec((2 * D, T), lambda i: (0, 0)),
            pl.BlockSpec((1, T), lambda i: (0, 0)),
        ],
        out_specs=pl.BlockSpec((B, T), lambda i: (0, 0)),
        out_shape=jax.ShapeDtypeStruct((B, T), _f32),
        scratch_shapes=[pltpu.VMEM((B, B), _f32)],
    )(o, b1, batchp, fW1, fb1, fW2, fb2)


def kernel(x, edge_index, edge_attr, batch, feat_table, W0, att_s0, att_d0, b0,
           W1, att_s1, att_d1, b1, fW1, fb1, fW2, fb2):
    n = x.shape[0]
    e = edge_index.shape[1]

    loop = jnp.arange(n, dtype=_i32)
    npad = EPAD - e - n
    src = jnp.concatenate([edge_index[0].astype(_i32), loop,
                           jnp.zeros((npad,), _i32)])
    dst = jnp.concatenate([edge_index[1].astype(_i32), loop,
                           jnp.full((npad,), NP, _i32)])
    xp = jnp.pad(x.astype(_i32), (0, NP - n))
    ftp = jnp.pad(feat_table, ((0, CP - 21), (0, 0)))

    T0p, s0t, d0t = _tc_prep(ftp, W0, att_s0.reshape(D, 1),
                             att_d0.reshape(D, 1))

    cflat = _get_sc_layer0()(xp, src, dst, s0t.reshape(CP), d0t.reshape(CP))
    cmat = cflat.reshape(NP, CP)

    H1, as1, ad1 = _tc_mid(cmat, T0p, b0.reshape(1, D), W1,
                           att_s1.reshape(D, 1), att_d1.reshape(D, 1))

    o1 = _get_sc_layer1()(H1, as1.reshape(NP), ad1.reshape(NP), src, dst)

    batchp = jnp.pad(batch.astype(_i32), (0, NP - n),
                     constant_values=B).reshape(NP, 1)
    return _tc_final(o1, b1.reshape(1, D), batchp, fW1,
                     fb1.reshape(1, 2 * D), fW2, fb2.reshape(1, T))


# speedup vs baseline: 50.3770x; 1.3082x over previous
import functools

import jax
import jax.numpy as jnp
from jax import lax
from jax.experimental import pallas as pl
from jax.experimental.pallas import tpu as pltpu
from jax.experimental.pallas import tpu_sc as plsc

N = 10000
NP = 10240
HALF = NP // 2
HPAD = 5248
D = 128
B = 128
T = 10
CP = 32
CH = 128
PK = 16384
EPAD = 344064
ETILE = EPAD // 16
ECAP = ETILE + CH
HSTRIPE = HPAD // 16

_f32 = jnp.float32
_i32 = jnp.int32


def _leaky(u):
    return jnp.where(u >= 0, u, 0.2 * u)


_GDN = lax.GatherDimensionNumbers(
    offset_dims=(), collapsed_slice_dims=(0,), start_index_map=(0,))


def _perm(v, idx):
    return lax.gather(v, idx[:, None], _GDN, (1,),
                      mode=lax.GatherScatterMode.PROMISE_IN_BOUNDS)


def _lane_max(v):
    idx = lax.iota(_i32, 16)
    for s2 in (8, 4, 2, 1):
        v = jnp.maximum(v, _perm(v, idx ^ s2))
    return v


def _sc_mesh():
    return plsc.VectorSubcoreMesh(core_axis_name="c", subcore_axis_name="s",
                                  num_cores=2, num_subcores=16)


@functools.lru_cache(maxsize=None)
def _get_sc_layer0():
    @functools.partial(
        pl.kernel,
        out_type=jax.ShapeDtypeStruct((NP * CP,), _f32),
        mesh=_sc_mesh(),
        compiler_params=pltpu.CompilerParams(needs_layout_passes=False),
        scratch_types=[
            pltpu.VMEM((NP,), _i32),
            pltpu.VMEM((CP,), _f32),
            pltpu.VMEM((CP,), _f32),
            pltpu.VMEM((ECAP,), _i32),
            pltpu.VMEM((ECAP,), _f32),
            pltpu.VMEM((HPAD,), _f32),
            pltpu.VMEM((2624,), _f32),
            pltpu.VMEM((2, CH), _i32),
            pltpu.VMEM_SHARED((HPAD,), _f32),
            pltpu.VMEM_SHARED((HPAD * CP,), _f32),
            pltpu.SemaphoreType.DMA,
        ],
    )
    def sc_layer0(xh, pkh, s0h, d0h, out_h,
                  x_v, s0_v, d0_v, pk1, ex1, den_v, zero_v, ix2,
                  sh_den, sh_c, sd):
        s = lax.axis_index("s")
        c = lax.axis_index("c")
        base = c * HALF

        pltpu.sync_copy(xh, x_v)
        pltpu.sync_copy(s0h, s0_v)
        pltpu.sync_copy(d0h, d0_v)
        pltpu.sync_copy(pkh.at[pl.ds(s * ETILE, ETILE)],
                        pk1.at[pl.ds(0, ETILE)])

        ms = _lane_max(jnp.maximum(s0_v[pl.ds(0, 16)], s0_v[pl.ds(16, 16)]))
        md = _lane_max(jnp.maximum(d0_v[pl.ds(0, 16)], d0_v[pl.ds(16, 16)]))
        shift = _leaky(ms + md)

        def _z(i, carry):
            zero_v[pl.ds(i * 16, 16)] = jnp.zeros((16,), _f32)
            return carry
        lax.fori_loop(0, 164, _z, 0)
        pltpu.sync_copy(zero_v.at[pl.ds(0, HSTRIPE)],
                        sh_den.at[pl.ds(s * HSTRIPE, HSTRIPE)])

        def _zc(t, carry):
            pltpu.sync_copy(
                zero_v, sh_c.at[pl.ds((s * 4 + t) * 2624, 2624)])
            return carry
        lax.fori_loop(0, 4, _zc, 0)
        plsc.subcore_barrier()

        def _scan(g, cnt):
            sl = pl.ds(g * 16, 16)
            p16 = pk1[sl]
            s16 = p16 & (PK - 1)
            d16 = jnp.right_shift(p16, 14)
            xs = plsc.load_gather(x_v, [s16])
            xd = plsc.load_gather(x_v, [jnp.minimum(d16, NP - 1)])
            a = plsc.load_gather(s0_v, [xs])
            b = plsc.load_gather(d0_v, [xd])
            ex = jnp.exp(_leaky(a + b) - shift)
            ld = d16 - base
            m = (ld >= 0) & (ld < HALF)
            plsc.store_compressed(pk1.at[pl.ds(cnt, 16)], ld * CP + xs,
                                  mask=m)
            plsc.store_compressed(ex1.at[pl.ds(cnt, 16)], ex, mask=m)
            return cnt + plsc.all_reduce_population_count(m)[0]
        cnt = lax.fori_loop(0, ETILE // 16, _scan, 0)
        ii = lax.iota(_i32, 16)
        for j in range(8):
            idx = cnt + j * 16 + ii
            plsc.store_scatter(pk1, [idx], (HALF + j * 16 + ii) * CP)
            plsc.store_scatter(ex1, [idx], jnp.zeros((16,), _f32))
        nk = (cnt + CH - 1) // CH

        def _den(kk, carry):
            b = kk % 2
            for j in range(8):
                slo = pl.ds(j * 16, 16)
                ix2[b, slo] = jnp.right_shift(
                    pk1[pl.ds(kk * CH + j * 16, 16)], 5)

            @pl.when(kk > 0)
            def _():
                pltpu.make_async_copy(ex1.at[pl.ds((kk - 1) * CH, CH)],
                                      sh_den.at[ix2.at[1 - b]], sd).wait()
            pltpu.async_copy(ex1.at[pl.ds(kk * CH, CH)],
                             sh_den.at[ix2.at[b]], sd, add=True)
            return carry
        lax.fori_loop(0, nk, _den, 0)

        @pl.when(nk > 0)
        def _():
            pltpu.make_async_copy(ex1.at[pl.ds((nk - 1) * CH, CH)],
                                  sh_den.at[ix2.at[(nk - 1) % 2]], sd).wait()
        plsc.subcore_barrier()

        pltpu.sync_copy(sh_den, den_v)

        def _hist(kk, carry):
            b = kk % 2
            for j in range(8):
                sl = pl.ds(kk * CH + j * 16, 16)
                hx = pk1[sl]
                den16 = plsc.load_gather(den_v, [jnp.right_shift(hx, 5)])
                ex1[sl] = ex1[sl] / (den16 + 1e-16)
                ix2[b, pl.ds(j * 16, 16)] = hx

            @pl.when(kk > 0)
            def _():
                pltpu.make_async_copy(ex1.at[pl.ds((kk - 1) * CH, CH)],
                                      sh_c.at[ix2.at[1 - b]], sd).wait()
            pltpu.async_copy(ex1.at[pl.ds(kk * CH, CH)],
                             sh_c.at[ix2.at[b]], sd, add=True)
            return carry
        lax.fori_loop(0, nk, _hist, 0)

        @pl.when(nk > 0)
        def _():
            pltpu.make_async_copy(ex1.at[pl.ds((nk - 1) * CH, CH)],
                                  sh_c.at[ix2.at[(nk - 1) % 2]], sd).wait()
        plsc.subcore_barrier()

        pltpu.sync_copy(
            sh_c.at[pl.ds(s * (HALF // 16) * CP, (HALF // 16) * CP)],
            out_h.at[pl.ds(c * HALF * CP + s * (HALF // 16) * CP,
                           (HALF // 16) * CP)])

    return sc_layer0


@functools.lru_cache(maxsize=None)
def _get_sc_layer1():
    @functools.partial(
        pl.kernel,
        out_type=jax.ShapeDtypeStruct((NP, D), _f32),
        mesh=_sc_mesh(),
        compiler_params=pltpu.CompilerParams(needs_layout_passes=False),
        scratch_types=[
            pltpu.VMEM((NP,), _f32),
            pltpu.VMEM((HPAD,), _f32),
            pltpu.VMEM((ECAP,), _i32),
            pltpu.VMEM((HPAD,), _f32),
            pltpu.VMEM((2 * CH, D), _f32),
            pltpu.VMEM((8, D), _f32),
            pltpu.VMEM((HSTRIPE,), _f32),
            pltpu.VMEM((2, CH), _i32),
            pltpu.VMEM((2, CH), _i32),
            pltpu.VMEM((2, CH), _f32),
            pltpu.VMEM_SHARED((HPAD,), _f32),
            pltpu.VMEM_SHARED((HPAD, D), _f32),
            pltpu.SemaphoreType.DMA,
            pltpu.SemaphoreType.DMA,
            pltpu.SemaphoreType.DMA,
        ],
    )
    def sc_layer1(hh, ash, adh, pkh, out_h,
                  as_v, adh_v, pk1, den_v, rows2, zrow_v, zden_v,
                  ix2, is2, ex2, sh_den, sh_out, sg, ss, sd):
        s = lax.axis_index("s")
        c = lax.axis_index("c")
        base = c * HALF

        pltpu.sync_copy(ash, as_v)
        pltpu.sync_copy(adh.at[pl.ds(base, HALF)], adh_v.at[pl.ds(0, HALF)])
        for j in range(8):
            adh_v[pl.ds(HALF + j * 16, 16)] = jnp.zeros((16,), _f32)
        pltpu.sync_copy(pkh.at[pl.ds(s * ETILE, ETILE)],
                        pk1.at[pl.ds(0, ETILE)])

        def _mx(ref, ng):
            def step(i, m):
                return jnp.maximum(m, ref[pl.ds(i * 16, 16)])
            return _lane_max(lax.fori_loop(0, ng, step,
                                           jnp.full((16,), -1e30, _f32)))
        ngroups = 305 + (1 - c) * 15
        shift = _leaky(_mx(as_v, N // 16) + _mx(adh_v, ngroups))

        def _z1(i, carry):
            zden_v[pl.ds(i * 16, 16)] = jnp.zeros((16,), _f32)
            return carry
        lax.fori_loop(0, HSTRIPE // 16, _z1, 0)

        def _z2(i, carry):
            for j in range(8):
                zrow_v[i, pl.ds(j * 16, 16)] = jnp.zeros((16,), _f32)
            return carry
        lax.fori_loop(0, 8, _z2, 0)
        pltpu.sync_copy(zden_v, sh_den.at[pl.ds(s * HSTRIPE, HSTRIPE)])

        def _z3(t, carry):
            pltpu.sync_copy(zrow_v,
                            sh_out.at[pl.ds(s * HSTRIPE + t * 8, 8)])
            return carry
        lax.fori_loop(0, HSTRIPE // 8, _z3, 0)
        plsc.subcore_barrier()

        def _scan(g, cnt):
            sl = pl.ds(g * 16, 16)
            p16 = pk1[sl]
            ld = jnp.right_shift(p16, 14) - base
            m = (ld >= 0) & (ld < HALF)
            plsc.store_compressed(pk1.at[pl.ds(cnt, 16)],
                                  p16 - base * PK, mask=m)
            return cnt + plsc.all_reduce_population_count(m)[0]
        cnt = lax.fori_loop(0, ETILE // 16, _scan, 0)
        ii = lax.iota(_i32, 16)
        for j in range(8):
            idx = cnt + j * 16 + ii
            plsc.store_scatter(pk1, [idx], (HALF + j * 16 + ii) * PK)
        nk = (cnt + CH - 1) // CH

        def _prep(kk, b):
            for j in range(8):
                slo = pl.ds(j * 16, 16)
                p16 = pk1[pl.ds(kk * CH + j * 16, 16)]
                ix2[b, slo] = jnp.right_shift(p16, 14)
                is2[b, slo] = p16 & (PK - 1)

        def _exw(b, j):
            slo = pl.ds(j * 16, 16)
            a = plsc.load_gather(as_v, [is2[b, slo]])
            bb = plsc.load_gather(adh_v, [ix2[b, slo]])
            return jnp.exp(_leaky(a + bb) - shift)

        def _den(kk, carry):
            b = kk % 2
            _prep(kk, b)
            for j in range(8):
                ex2[b, pl.ds(j * 16, 16)] = _exw(b, j)

            @pl.when(kk > 0)
            def _():
                pltpu.make_async_copy(ex2.at[1 - b],
                                      sh_den.at[ix2.at[1 - b]], sd).wait()
            pltpu.async_copy(ex2.at[b], sh_den.at[ix2.at[b]], sd, add=True)
            return carry
        lax.fori_loop(0, nk, _den, 0)

        @pl.when(nk > 0)
        def _():
            lb = (nk - 1) % 2
            pltpu.make_async_copy(ex2.at[lb], sh_den.at[ix2.at[lb]],
                                  sd).wait()
        plsc.subcore_barrier()

        pltpu.sync_copy(sh_den, den_v)

        @pl.when(nk > 0)
        def _():
            _prep(0, 0)
            pltpu.async_copy(hh.at[is2.at[0]], rows2.at[pl.ds(0, CH)], sg)

        def _rows(kk, carry):
            b = kk % 2
            nb = 1 - b

            @pl.when(kk >= 1)
            def _():
                pltpu.make_async_copy(rows2.at[pl.ds(nb * CH, CH)],
                                      sh_out.at[ix2.at[nb]], ss).wait()

            @pl.when(kk + 1 < nk)
            def _():
                _prep(kk + 1, nb)
                pltpu.async_copy(hh.at[is2.at[nb]],
                                 rows2.at[pl.ds(nb * CH, CH)], sg)

            for j in range(8):
                slo = pl.ds(j * 16, 16)
                den16 = plsc.load_gather(den_v, [ix2[b, slo]])
                ex2[b, slo] = _exw(b, j) / (den16 + 1e-16)
            pltpu.make_async_copy(hh.at[is2.at[b]],
                                  rows2.at[pl.ds(b * CH, CH)], sg).wait()

            for j in range(8):
                coef16 = ex2[b, pl.ds(j * 16, 16)]
                for l in range(16):
                    cc = coef16[l]
                    e = b * CH + j * 16 + l
                    for jj in range(8):
                        sl2 = pl.ds(jj * 16, 16)
                        rows2[e, sl2] = rows2[e, sl2] * cc
            pltpu.async_copy(rows2.at[pl.ds(b * CH, CH)],
                             sh_out.at[ix2.at[b]], ss, add=True)
            return carry
        lax.fori_loop(0, nk, _rows, 0)

        @pl.when(nk > 0)
        def _():
            lb = (nk - 1) % 2
            pltpu.make_async_copy(rows2.at[pl.ds(lb * CH, CH)],
                                  sh_out.at[ix2.at[lb]], ss).wait()
        plsc.subcore_barrier()

        pltpu.sync_copy(sh_out.at[pl.ds(s * (HALF // 16), HALF // 16)],
                        out_h.at[pl.ds(c * HALF + s * (HALF // 16),
                                       HALF // 16)])

    return sc_layer1


def _tc_prep(ftp, W0, asw, adw):
    def body(ft_r, w_r, as_r, ad_r, t0_r, s_r, d_r):
        t0 = jnp.dot(ft_r[...], w_r[...], preferred_element_type=_f32)
        t0_r[...] = t0
        rid = lax.broadcasted_iota(_i32, (CP, 1), 0)
        sval = jnp.dot(t0, as_r[...], preferred_element_type=_f32)
        dval = jnp.dot(t0, ad_r[...], preferred_element_type=_f32)
        s_r[...] = jnp.where(rid < 21, sval, -1e30)
        d_r[...] = jnp.where(rid < 21, dval, -1e30)

    return pl.pallas_call(
        body,
        out_shape=[
            jax.ShapeDtypeStruct((CP, D), _f32),
            jax.ShapeDtypeStruct((CP, 1), _f32),
            jax.ShapeDtypeStruct((CP, 1), _f32),
        ],
    )(ftp, W0, asw, adw)


_R = 1024


def _tc_mid(cmat, T0p, b0, W1, asw, adw):
    def body(c_r, t0_r, b0_r, w1_r, as_r, ad_r, h_r, s_r, d_r):
        h1 = jnp.maximum(
            jnp.dot(c_r[...], t0_r[...], preferred_element_type=_f32)
            + b0_r[...], 0.0)
        H1 = jnp.dot(h1, w1_r[...], preferred_element_type=_f32)
        h_r[...] = H1
        s_r[...] = jnp.dot(H1, as_r[...], preferred_element_type=_f32)
        d_r[...] = jnp.dot(H1, ad_r[...], preferred_element_type=_f32)

    grid = (NP // _R,)
    return pl.pallas_call(
        body,
        grid=grid,
        in_specs=[
            pl.BlockSpec((_R, CP), lambda i: (i, 0)),
            pl.BlockSpec((CP, D), lambda i: (0, 0)),
            pl.BlockSpec((1, D), lambda i: (0, 0)),
            pl.BlockSpec((D, D), lambda i: (0, 0)),
            pl.BlockSpec((D, 1), lambda i: (0, 0)),
            pl.BlockSpec((D, 1), lambda i: (0, 0)),
        ],
        out_specs=[
            pl.BlockSpec((_R, D), lambda i: (i, 0)),
            pl.BlockSpec((_R, 1), lambda i: (i, 0)),
            pl.BlockSpec((_R, 1), lambda i: (i, 0)),
        ],
        out_shape=[
            jax.ShapeDtypeStruct((NP, D), _f32),
            jax.ShapeDtypeStruct((NP, 1), _f32),
            jax.ShapeDtypeStruct((NP, 1), _f32),
        ],
    )(cmat, T0p, b0, W1, asw, adw)


def _tc_final(o, b1, batchp, fW1, fb1, fW2, fb2):
    def body(o_r, b1_r, bt_r, w1_r, q1_r, w2_r, q2_r, out_r, g_acc):
        i = pl.program_id(0)

        @pl.when(i == 0)
        def _():
            g_acc[...] = jnp.zeros_like(g_acc)

        h2 = jnp.maximum(o_r[...] + b1_r[...], 0.0)
        col = lax.broadcasted_iota(_i32, (_R, B), 1)
        oh = (bt_r[...] == col).astype(_f32)
        g_acc[...] += lax.dot_general(
            oh, h2, (((0,), (0,)), ((), ())), preferred_element_type=_f32)

        @pl.when(i == NP // _R - 1)
        def _():
            g = g_acc[...]
            z = jnp.maximum(
                jnp.dot(g, w1_r[...], preferred_element_type=_f32) + q1_r[...],
                0.0)
            out_r[...] = (jnp.dot(z, w2_r[...], preferred_element_type=_f32)
                          + q2_r[...])

    grid = (NP // _R,)
    return pl.pallas_call(
        body,
        grid=grid,
        in_specs=[
            pl.BlockSpec((_R, D), lambda i: (i, 0)),
            pl.BlockSpec((1, D), lambda i: (0, 0)),
            pl.BlockSpec((_R, 1), lambda i: (i, 0)),
            pl.BlockSpec((D, 2 * D), lambda i: (0, 0)),
            pl.BlockSpec((1, 2 * D), lambda i: (0, 0)),
            pl.BlockSpec((2 * D, T), lambda i: (0, 0)),
            pl.BlockSpec((1, T), lambda i: (0, 0)),
        ],
        out_specs=pl.BlockSpec((B, T), lambda i: (0, 0)),
        out_shape=jax.ShapeDtypeStruct((B, T), _f32),
        scratch_shapes=[pltpu.VMEM((B, B), _f32)],
    )(o, b1, batchp, fW1, fb1, fW2, fb2)


def kernel(x, edge_index, edge_attr, batch, feat_table, W0, att_s0, att_d0, b0,
           W1, att_s1, att_d1, b1, fW1, fb1, fW2, fb2):
    n = x.shape[0]
    e = edge_index.shape[1]

    loop = jnp.arange(n, dtype=_i32)
    npad = EPAD - e - n
    src = jnp.concatenate([edge_index[0].astype(_i32), loop,
                           jnp.zeros((npad,), _i32)])
    dst = jnp.concatenate([edge_index[1].astype(_i32), loop,
                           jnp.full((npad,), NP, _i32)])
    pk = dst * PK + src
    xp = jnp.pad(x.astype(_i32), (0, NP - n))
    ftp = jnp.pad(feat_table, ((0, CP - 21), (0, 0)))

    T0p, s0t, d0t = _tc_prep(ftp, W0, att_s0.reshape(D, 1),
                             att_d0.reshape(D, 1))

    cflat = _get_sc_layer0()(xp, pk, s0t.reshape(CP), d0t.reshape(CP))
    cmat = cflat.reshape(NP, CP)

    H1, as1, ad1 = _tc_mid(cmat, T0p, b0.reshape(1, D), W1,
                           att_s1.reshape(D, 1), att_d1.reshape(D, 1))

    o1 = _get_sc_layer1()(H1, as1.reshape(NP), ad1.reshape(NP), pk)

    batchp = jnp.pad(batch.astype(_i32), (0, NP - n),
                     constant_values=B).reshape(NP, 1)
    return _tc_final(o1, b1.reshape(1, D), batchp, fW1,
                     fb1.reshape(1, 2 * D), fW2, fb2.reshape(1, T))


# speedup vs baseline: 55.4613x; 1.1009x over previous
import functools

import jax
import jax.numpy as jnp
from jax import lax
from jax.experimental import pallas as pl
from jax.experimental.pallas import tpu as pltpu
from jax.experimental.pallas import tpu_sc as plsc

N = 10000
NP = 10240
HALF = NP // 2
HPAD = 5248
D = 128
B = 128
T = 10
CP = 32
CH = 128
CHR = 96
PK = 16384
EPAD = 344064
ETILE = EPAD // 16
ECAP = ETILE + CH
HSTRIPE = HPAD // 16

_f32 = jnp.float32
_i32 = jnp.int32


def _leaky(u):
    return jnp.where(u >= 0, u, 0.2 * u)


_GDN = lax.GatherDimensionNumbers(
    offset_dims=(), collapsed_slice_dims=(0,), start_index_map=(0,))


def _perm(v, idx):
    return lax.gather(v, idx[:, None], _GDN, (1,),
                      mode=lax.GatherScatterMode.PROMISE_IN_BOUNDS)


def _lane_max(v):
    idx = lax.iota(_i32, 16)
    for s2 in (8, 4, 2, 1):
        v = jnp.maximum(v, _perm(v, idx ^ s2))
    return v


def _sc_mesh():
    return plsc.VectorSubcoreMesh(core_axis_name="c", subcore_axis_name="s",
                                  num_cores=2, num_subcores=16)


@functools.lru_cache(maxsize=None)
def _get_sc_layer0():
    @functools.partial(
        pl.kernel,
        out_type=jax.ShapeDtypeStruct((NP * CP,), _f32),
        mesh=_sc_mesh(),
        compiler_params=pltpu.CompilerParams(needs_layout_passes=False),
        scratch_types=[
            pltpu.VMEM((NP,), _i32),
            pltpu.VMEM((CP,), _f32),
            pltpu.VMEM((CP,), _f32),
            pltpu.VMEM((ECAP,), _i32),
            pltpu.VMEM((ECAP,), _f32),
            pltpu.VMEM((HPAD,), _f32),
            pltpu.VMEM((2624,), _f32),
            pltpu.VMEM((2, CH), _i32),
            pltpu.VMEM_SHARED((HPAD,), _f32),
            pltpu.VMEM_SHARED((HPAD * CP,), _f32),
            pltpu.SemaphoreType.DMA,
        ],
    )
    def sc_layer0(xh, pkh, s0h, d0h, out_h,
                  x_v, s0_v, d0_v, pk1, ex1, den_v, zero_v, ix2,
                  sh_den, sh_c, sd):
        s = lax.axis_index("s")
        c = lax.axis_index("c")
        base = c * HALF

        pltpu.sync_copy(xh, x_v)
        pltpu.sync_copy(s0h, s0_v)
        pltpu.sync_copy(d0h, d0_v)
        pltpu.sync_copy(pkh.at[pl.ds(s * ETILE, ETILE)],
                        pk1.at[pl.ds(0, ETILE)])

        ms = _lane_max(jnp.maximum(s0_v[pl.ds(0, 16)], s0_v[pl.ds(16, 16)]))
        md = _lane_max(jnp.maximum(d0_v[pl.ds(0, 16)], d0_v[pl.ds(16, 16)]))
        shift = _leaky(ms + md)

        def _z(i, carry):
            zero_v[pl.ds(i * 16, 16)] = jnp.zeros((16,), _f32)
            return carry
        lax.fori_loop(0, 164, _z, 0)
        pltpu.sync_copy(zero_v.at[pl.ds(0, HSTRIPE)],
                        sh_den.at[pl.ds(s * HSTRIPE, HSTRIPE)])

        def _zc(t, carry):
            pltpu.sync_copy(
                zero_v, sh_c.at[pl.ds((s * 4 + t) * 2624, 2624)])
            return carry
        lax.fori_loop(0, 4, _zc, 0)
        plsc.subcore_barrier()

        def _scan(g, cnt):
            sl = pl.ds(g * 16, 16)
            p16 = pk1[sl]
            s16 = p16 & (PK - 1)
            d16 = jnp.right_shift(p16, 14)
            xs = plsc.load_gather(x_v, [s16])
            xd = plsc.load_gather(x_v, [jnp.minimum(d16, NP - 1)])
            a = plsc.load_gather(s0_v, [xs])
            b = plsc.load_gather(d0_v, [xd])
            ex = jnp.exp(_leaky(a + b) - shift)
            ld = d16 - base
            m = (ld >= 0) & (ld < HALF)
            plsc.store_compressed(pk1.at[pl.ds(cnt, 16)], ld * CP + xs,
                                  mask=m)
            plsc.store_compressed(ex1.at[pl.ds(cnt, 16)], ex, mask=m)
            return cnt + plsc.all_reduce_population_count(m)[0]
        cnt = lax.fori_loop(0, ETILE // 16, _scan, 0)
        ii = lax.iota(_i32, 16)
        for j in range(8):
            idx = cnt + j * 16 + ii
            plsc.store_scatter(pk1, [idx], (HALF + j * 16 + ii) * CP)
            plsc.store_scatter(ex1, [idx], jnp.zeros((16,), _f32))
        nk = (cnt + CH - 1) // CH

        def _den(kk, carry):
            b = kk % 2
            for j in range(8):
                slo = pl.ds(j * 16, 16)
                ix2[b, slo] = jnp.right_shift(
                    pk1[pl.ds(kk * CH + j * 16, 16)], 5)

            @pl.when(kk > 0)
            def _():
                pltpu.make_async_copy(ex1.at[pl.ds((kk - 1) * CH, CH)],
                                      sh_den.at[ix2.at[1 - b]], sd).wait()
            pltpu.async_copy(ex1.at[pl.ds(kk * CH, CH)],
                             sh_den.at[ix2.at[b]], sd, add=True)
            return carry
        lax.fori_loop(0, nk, _den, 0)

        @pl.when(nk > 0)
        def _():
            pltpu.make_async_copy(ex1.at[pl.ds((nk - 1) * CH, CH)],
                                  sh_den.at[ix2.at[(nk - 1) % 2]], sd).wait()
        plsc.subcore_barrier()

        pltpu.sync_copy(sh_den, den_v)

        def _hist(kk, carry):
            b = kk % 2
            for j in range(8):
                sl = pl.ds(kk * CH + j * 16, 16)
                hx = pk1[sl]
                den16 = plsc.load_gather(den_v, [jnp.right_shift(hx, 5)])
                ex1[sl] = ex1[sl] / (den16 + 1e-16)
                ix2[b, pl.ds(j * 16, 16)] = hx

            @pl.when(kk > 0)
            def _():
                pltpu.make_async_copy(ex1.at[pl.ds((kk - 1) * CH, CH)],
                                      sh_c.at[ix2.at[1 - b]], sd).wait()
            pltpu.async_copy(ex1.at[pl.ds(kk * CH, CH)],
                             sh_c.at[ix2.at[b]], sd, add=True)
            return carry
        lax.fori_loop(0, nk, _hist, 0)

        @pl.when(nk > 0)
        def _():
            pltpu.make_async_copy(ex1.at[pl.ds((nk - 1) * CH, CH)],
                                  sh_c.at[ix2.at[(nk - 1) % 2]], sd).wait()
        plsc.subcore_barrier()

        pltpu.sync_copy(
            sh_c.at[pl.ds(s * (HALF // 16) * CP, (HALF // 16) * CP)],
            out_h.at[pl.ds(c * HALF * CP + s * (HALF // 16) * CP,
                           (HALF // 16) * CP)])

    return sc_layer0


@functools.lru_cache(maxsize=None)
def _get_sc_layer1():
    @functools.partial(
        pl.kernel,
        out_type=jax.ShapeDtypeStruct((NP, D), _f32),
        mesh=_sc_mesh(),
        compiler_params=pltpu.CompilerParams(needs_layout_passes=False),
        scratch_types=[
            pltpu.VMEM((NP,), _f32),
            pltpu.VMEM((HPAD,), _f32),
            pltpu.VMEM((ECAP,), _i32),
            pltpu.VMEM((HPAD,), _f32),
            pltpu.VMEM((3 * CHR, D), _f32),
            pltpu.VMEM((8, D), _f32),
            pltpu.VMEM((HSTRIPE,), _f32),
            pltpu.VMEM((3, CHR), _i32),
            pltpu.VMEM((3, CHR), _i32),
            pltpu.VMEM((3, CHR), _f32),
            pltpu.VMEM_SHARED((HPAD,), _f32),
            pltpu.VMEM_SHARED((HPAD, D), _f32),
            pltpu.SemaphoreType.DMA,
            pltpu.SemaphoreType.DMA,
            pltpu.SemaphoreType.DMA,
        ],
    )
    def sc_layer1(hh, ash, adh, pkh, out_h,
                  as_v, adh_v, pk1, den_v, rows3, zrow_v, zden_v,
                  ix2, is2, ex2, sh_den, sh_out, sg, ss, sd):
        s = lax.axis_index("s")
        c = lax.axis_index("c")
        base = c * HALF

        pltpu.sync_copy(ash, as_v)
        pltpu.sync_copy(adh.at[pl.ds(base, HALF)], adh_v.at[pl.ds(0, HALF)])
        for j in range(8):
            adh_v[pl.ds(HALF + j * 16, 16)] = jnp.zeros((16,), _f32)
        pltpu.sync_copy(pkh.at[pl.ds(s * ETILE, ETILE)],
                        pk1.at[pl.ds(0, ETILE)])

        def _mx(ref, ng):
            def step(i, m):
                return jnp.maximum(m, ref[pl.ds(i * 16, 16)])
            return _lane_max(lax.fori_loop(0, ng, step,
                                           jnp.full((16,), -1e30, _f32)))
        ngroups = 305 + (1 - c) * 15
        shift = _leaky(_mx(as_v, N // 16) + _mx(adh_v, ngroups))

        def _z1(i, carry):
            zden_v[pl.ds(i * 16, 16)] = jnp.zeros((16,), _f32)
            return carry
        lax.fori_loop(0, HSTRIPE // 16, _z1, 0)

        def _z2(i, carry):
            for j in range(8):
                zrow_v[i, pl.ds(j * 16, 16)] = jnp.zeros((16,), _f32)
            return carry
        lax.fori_loop(0, 8, _z2, 0)
        pltpu.sync_copy(zden_v, sh_den.at[pl.ds(s * HSTRIPE, HSTRIPE)])

        def _z3(t, carry):
            pltpu.sync_copy(zrow_v,
                            sh_out.at[pl.ds(s * HSTRIPE + t * 8, 8)])
            return carry
        lax.fori_loop(0, HSTRIPE // 8, _z3, 0)
        plsc.subcore_barrier()

        def _scan(g, cnt):
            sl = pl.ds(g * 16, 16)
            p16 = pk1[sl]
            ld = jnp.right_shift(p16, 14) - base
            m = (ld >= 0) & (ld < HALF)
            plsc.store_compressed(pk1.at[pl.ds(cnt, 16)],
                                  p16 - base * PK, mask=m)
            return cnt + plsc.all_reduce_population_count(m)[0]
        cnt = lax.fori_loop(0, ETILE // 16, _scan, 0)
        ii = lax.iota(_i32, 16)
        for j in range(CHR // 16):
            idx = cnt + j * 16 + ii
            plsc.store_scatter(pk1, [idx], (HALF + j * 16 + ii) * PK)
        nk = (cnt + CHR - 1) // CHR

        def _prep(kk, b):
            for j in range(CHR // 16):
                slo = pl.ds(j * 16, 16)
                p16 = pk1[pl.ds(kk * CHR + j * 16, 16)]
                ix2[b, slo] = jnp.right_shift(p16, 14)
                is2[b, slo] = p16 & (PK - 1)

        def _exw(b, j):
            slo = pl.ds(j * 16, 16)
            a = plsc.load_gather(as_v, [is2[b, slo]])
            bb = plsc.load_gather(adh_v, [ix2[b, slo]])
            return jnp.exp(_leaky(a + bb) - shift)

        def _den(kk, carry):
            b = kk % 3
            nb = (kk + 1) % 3

            @pl.when(kk >= 2)
            def _():
                pltpu.make_async_copy(ex2.at[nb], sh_den.at[ix2.at[nb]],
                                      sd).wait()
            _prep(kk, b)
            for j in range(CHR // 16):
                ex2[b, pl.ds(j * 16, 16)] = _exw(b, j)
            pltpu.async_copy(ex2.at[b], sh_den.at[ix2.at[b]], sd, add=True)
            return carry
        lax.fori_loop(0, nk, _den, 0)

        for back in (2, 1):
            @pl.when(nk >= back)
            def _(back=back):
                lb = (nk - back) % 3
                pltpu.make_async_copy(ex2.at[lb], sh_den.at[ix2.at[lb]],
                                      sd).wait()
        plsc.subcore_barrier()

        pltpu.sync_copy(sh_den, den_v)

        @pl.when(nk > 0)
        def _():
            _prep(0, 0)
            pltpu.async_copy(hh.at[is2.at[0]], rows3.at[pl.ds(0, CHR)], sg)

        def _rows(kk, carry):
            b = kk % 3
            nb = (kk + 1) % 3

            @pl.when(kk >= 2)
            def _():
                pltpu.make_async_copy(rows3.at[pl.ds(nb * CHR, CHR)],
                                      sh_out.at[ix2.at[nb]], ss).wait()

            @pl.when(kk + 1 < nk)
            def _():
                _prep(kk + 1, nb)
                pltpu.async_copy(hh.at[is2.at[nb]],
                                 rows3.at[pl.ds(nb * CHR, CHR)], sg)

            for j in range(CHR // 16):
                slo = pl.ds(j * 16, 16)
                den16 = plsc.load_gather(den_v, [ix2[b, slo]])
                ex2[b, slo] = _exw(b, j) / (den16 + 1e-16)
            pltpu.make_async_copy(hh.at[is2.at[b]],
                                  rows3.at[pl.ds(b * CHR, CHR)], sg).wait()

            for j in range(CHR // 16):
                coef16 = ex2[b, pl.ds(j * 16, 16)]
                for l in range(16):
                    cc = coef16[l]
                    e = b * CHR + j * 16 + l
                    for jj in range(8):
                        sl2 = pl.ds(jj * 16, 16)
                        rows3[e, sl2] = rows3[e, sl2] * cc
            pltpu.async_copy(rows3.at[pl.ds(b * CHR, CHR)],
                             sh_out.at[ix2.at[b]], ss, add=True)
            return carry
        lax.fori_loop(0, nk, _rows, 0)

        for back in (2, 1):
            @pl.when(nk >= back)
            def _(back=back):
                lb = (nk - back) % 3
                pltpu.make_async_copy(rows3.at[pl.ds(lb * CHR, CHR)],
                                      sh_out.at[ix2.at[lb]], ss).wait()
        plsc.subcore_barrier()

        pltpu.sync_copy(sh_out.at[pl.ds(s * (HALF // 16), HALF // 16)],
                        out_h.at[pl.ds(c * HALF + s * (HALF // 16),
                                       HALF // 16)])

    return sc_layer1


def _tc_prep(ftp, W0, asw, adw):
    def body(ft_r, w_r, as_r, ad_r, t0_r, s_r, d_r):
        t0 = jnp.dot(ft_r[...], w_r[...], preferred_element_type=_f32)
        t0_r[...] = t0
        rid = lax.broadcasted_iota(_i32, (CP, 1), 0)
        sval = jnp.dot(t0, as_r[...], preferred_element_type=_f32)
        dval = jnp.dot(t0, ad_r[...], preferred_element_type=_f32)
        s_r[...] = jnp.where(rid < 21, sval, -1e30)
        d_r[...] = jnp.where(rid < 21, dval, -1e30)

    return pl.pallas_call(
        body,
        out_shape=[
            jax.ShapeDtypeStruct((CP, D), _f32),
            jax.ShapeDtypeStruct((CP, 1), _f32),
            jax.ShapeDtypeStruct((CP, 1), _f32),
        ],
    )(ftp, W0, asw, adw)


_R = 1024


def _tc_mid(cmat, T0p, b0, W1, asw, adw):
    def body(c_r, t0_r, b0_r, w1_r, as_r, ad_r, h_r, s_r, d_r):
        h1 = jnp.maximum(
            jnp.dot(c_r[...], t0_r[...], preferred_element_type=_f32)
            + b0_r[...], 0.0)
        H1 = jnp.dot(h1, w1_r[...], preferred_element_type=_f32)
        h_r[...] = H1
        s_r[...] = jnp.dot(H1, as_r[...], preferred_element_type=_f32)
        d_r[...] = jnp.dot(H1, ad_r[...], preferred_element_type=_f32)

    grid = (NP // _R,)
    return pl.pallas_call(
        body,
        grid=grid,
        in_specs=[
            pl.BlockSpec((_R, CP), lambda i: (i, 0)),
            pl.BlockSpec((CP, D), lambda i: (0, 0)),
            pl.BlockSpec((1, D), lambda i: (0, 0)),
            pl.BlockSpec((D, D), lambda i: (0, 0)),
            pl.BlockSpec((D, 1), lambda i: (0, 0)),
            pl.BlockSpec((D, 1), lambda i: (0, 0)),
        ],
        out_specs=[
            pl.BlockSpec((_R, D), lambda i: (i, 0)),
            pl.BlockSpec((_R, 1), lambda i: (i, 0)),
            pl.BlockSpec((_R, 1), lambda i: (i, 0)),
        ],
        out_shape=[
            jax.ShapeDtypeStruct((NP, D), _f32),
            jax.ShapeDtypeStruct((NP, 1), _f32),
            jax.ShapeDtypeStruct((NP, 1), _f32),
        ],
    )(cmat, T0p, b0, W1, asw, adw)


def _tc_final(o, b1, batchp, fW1, fb1, fW2, fb2):
    def body(o_r, b1_r, bt_r, w1_r, q1_r, w2_r, q2_r, out_r, g_acc):
        i = pl.program_id(0)

        @pl.when(i == 0)
        def _():
            g_acc[...] = jnp.zeros_like(g_acc)

        h2 = jnp.maximum(o_r[...] + b1_r[...], 0.0)
        col = lax.broadcasted_iota(_i32, (_R, B), 1)
        oh = (bt_r[...] == col).astype(_f32)
        g_acc[...] += lax.dot_general(
            oh, h2, (((0,), (0,)), ((), ())), preferred_element_type=_f32)

        @pl.when(i == NP // _R - 1)
        def _():
            g = g_acc[...]
            z = jnp.maximum(
                jnp.dot(g, w1_r[...], preferred_element_type=_f32) + q1_r[...],
                0.0)
            out_r[...] = (jnp.dot(z, w2_r[...], preferred_element_type=_f32)
                          + q2_r[...])

    grid = (NP // _R,)
    return pl.pallas_call(
        body,
        grid=grid,
        in_specs=[
            pl.BlockSpec((_R, D), lambda i: (i, 0)),
            pl.BlockSpec((1, D), lambda i: (0, 0)),
            pl.BlockSpec((_R, 1), lambda i: (i, 0)),
            pl.BlockSpec((D, 2 * D), lambda i: (0, 0)),
            pl.BlockSpec((1, 2 * D), lambda i: (0, 0)),
            pl.BlockSpec((2 * D, T), lambda i: (0, 0)),
            pl.BlockSpec((1, T), lambda i: (0, 0)),
        ],
        out_specs=pl.BlockSpec((B, T), lambda i: (0, 0)),
        out_shape=jax.ShapeDtypeStruct((B, T), _f32),
        scratch_shapes=[pltpu.VMEM((B, B), _f32)],
    )(o, b1, batchp, fW1, fb1, fW2, fb2)


def kernel(x, edge_index, edge_attr, batch, feat_table, W0, att_s0, att_d0, b0,
           W1, att_s1, att_d1, b1, fW1, fb1, fW2, fb2):
    n = x.shape[0]
    e = edge_index.shape[1]

    loop = jnp.arange(n, dtype=_i32)
    npad = EPAD - e - n
    src = jnp.concatenate([edge_index[0].astype(_i32), loop,
                           jnp.zeros((npad,), _i32)])
    dst = jnp.concatenate([edge_index[1].astype(_i32), loop,
                           jnp.full((npad,), NP, _i32)])
    pk = dst * PK + src
    xp = jnp.pad(x.astype(_i32), (0, NP - n))
    ftp = jnp.pad(feat_table, ((0, CP - 21), (0, 0)))

    T0p, s0t, d0t = _tc_prep(ftp, W0, att_s0.reshape(D, 1),
                             att_d0.reshape(D, 1))

    cflat = _get_sc_layer0()(xp, pk, s0t.reshape(CP), d0t.reshape(CP))
    cmat = cflat.reshape(NP, CP)

    H1, as1, ad1 = _tc_mid(cmat, T0p, b0.reshape(1, D), W1,
                           att_s1.reshape(D, 1), att_d1.reshape(D, 1))

    o1 = _get_sc_layer1()(H1, as1.reshape(NP), ad1.reshape(NP), pk)

    batchp = jnp.pad(batch.astype(_i32), (0, NP - n),
                     constant_values=B).reshape(NP, 1)
    return _tc_final(o1, b1.reshape(1, D), batchp, fW1,
                     fb1.reshape(1, 2 * D), fW2, fb2.reshape(1, T))


# speedup vs baseline: 55.7569x; 1.0053x over previous
import functools

import jax
import jax.numpy as jnp
from jax import lax
from jax.experimental import pallas as pl
from jax.experimental.pallas import tpu as pltpu
from jax.experimental.pallas import tpu_sc as plsc

N = 10000
NP = 10240
HALF = NP // 2
HPAD = 5248
D = 128
B = 128
T = 10
CP = 32
CH = 128
CHR = 96
PK = 16384
EPAD = 344064
ETILE = EPAD // 16
ECAP = ETILE + CH
HSTRIPE = HPAD // 16

_f32 = jnp.float32
_i32 = jnp.int32


def _leaky(u):
    return jnp.where(u >= 0, u, 0.2 * u)


_GDN = lax.GatherDimensionNumbers(
    offset_dims=(), collapsed_slice_dims=(0,), start_index_map=(0,))


def _perm(v, idx):
    return lax.gather(v, idx[:, None], _GDN, (1,),
                      mode=lax.GatherScatterMode.PROMISE_IN_BOUNDS)


def _lane_max(v):
    idx = lax.iota(_i32, 16)
    for s2 in (8, 4, 2, 1):
        v = jnp.maximum(v, _perm(v, idx ^ s2))
    return v


def _sc_mesh():
    return plsc.VectorSubcoreMesh(core_axis_name="c", subcore_axis_name="s",
                                  num_cores=2, num_subcores=16)


@functools.lru_cache(maxsize=None)
def _get_sc_layer0():
    @functools.partial(
        pl.kernel,
        out_type=jax.ShapeDtypeStruct((NP * CP,), _f32),
        mesh=_sc_mesh(),
        compiler_params=pltpu.CompilerParams(needs_layout_passes=False),
        scratch_types=[
            pltpu.VMEM((NP,), _i32),
            pltpu.VMEM((CP,), _f32),
            pltpu.VMEM((CP,), _f32),
            pltpu.VMEM((ECAP,), _i32),
            pltpu.VMEM((ECAP,), _f32),
            pltpu.VMEM((HPAD,), _f32),
            pltpu.VMEM((2624,), _f32),
            pltpu.VMEM((3, CH), _i32),
            pltpu.VMEM_SHARED((HPAD,), _f32),
            pltpu.VMEM_SHARED((HPAD * CP,), _f32),
            pltpu.SemaphoreType.DMA,
        ],
    )
    def sc_layer0(xh, pkh, s0h, d0h, out_h,
                  x_v, s0_v, d0_v, pk1, ex1, den_v, zero_v, ix2,
                  sh_den, sh_c, sd):
        s = lax.axis_index("s")
        c = lax.axis_index("c")
        base = c * HALF

        pltpu.sync_copy(xh, x_v)
        pltpu.sync_copy(s0h, s0_v)
        pltpu.sync_copy(d0h, d0_v)
        pltpu.sync_copy(pkh.at[pl.ds(s * ETILE, ETILE)],
                        pk1.at[pl.ds(0, ETILE)])

        ms = _lane_max(jnp.maximum(s0_v[pl.ds(0, 16)], s0_v[pl.ds(16, 16)]))
        md = _lane_max(jnp.maximum(d0_v[pl.ds(0, 16)], d0_v[pl.ds(16, 16)]))
        shift = _leaky(ms + md)

        def _z(i, carry):
            zero_v[pl.ds(i * 16, 16)] = jnp.zeros((16,), _f32)
            return carry
        lax.fori_loop(0, 164, _z, 0)
        pltpu.sync_copy(zero_v.at[pl.ds(0, HSTRIPE)],
                        sh_den.at[pl.ds(s * HSTRIPE, HSTRIPE)])

        def _zc(t, carry):
            pltpu.sync_copy(
                zero_v, sh_c.at[pl.ds((s * 4 + t) * 2624, 2624)])
            return carry
        lax.fori_loop(0, 4, _zc, 0)
        plsc.subcore_barrier()

        def _scan(g, cnt):
            sl = pl.ds(g * 16, 16)
            p16 = pk1[sl]
            s16 = p16 & (PK - 1)
            d16 = jnp.right_shift(p16, 14)
            xs = plsc.load_gather(x_v, [s16])
            xd = plsc.load_gather(x_v, [jnp.minimum(d16, NP - 1)])
            a = plsc.load_gather(s0_v, [xs])
            b = plsc.load_gather(d0_v, [xd])
            ex = jnp.exp(_leaky(a + b) - shift)
            ld = d16 - base
            m = (ld >= 0) & (ld < HALF)
            plsc.store_compressed(pk1.at[pl.ds(cnt, 16)], ld * CP + xs,
                                  mask=m)
            plsc.store_compressed(ex1.at[pl.ds(cnt, 16)], ex, mask=m)
            return cnt + plsc.all_reduce_population_count(m)[0]
        cnt = lax.fori_loop(0, ETILE // 16, _scan, 0)
        ii = lax.iota(_i32, 16)
        for j in range(8):
            idx = cnt + j * 16 + ii
            plsc.store_scatter(pk1, [idx], (HALF + j * 16 + ii) * CP)
            plsc.store_scatter(ex1, [idx], jnp.zeros((16,), _f32))
        nk = (cnt + CH - 1) // CH

        def _den(kk, carry):
            b = kk % 3
            nb = (kk + 1) % 3

            @pl.when(kk >= 2)
            def _():
                pltpu.make_async_copy(ex1.at[pl.ds((kk - 2) * CH, CH)],
                                      sh_den.at[ix2.at[nb]], sd).wait()
            for j in range(8):
                slo = pl.ds(j * 16, 16)
                ix2[b, slo] = jnp.right_shift(
                    pk1[pl.ds(kk * CH + j * 16, 16)], 5)
            pltpu.async_copy(ex1.at[pl.ds(kk * CH, CH)],
                             sh_den.at[ix2.at[b]], sd, add=True)
            return carry
        lax.fori_loop(0, nk, _den, 0)

        for back in (2, 1):
            @pl.when(nk >= back)
            def _(back=back):
                lb = (nk - back) % 3
                pltpu.make_async_copy(ex1.at[pl.ds((nk - back) * CH, CH)],
                                      sh_den.at[ix2.at[lb]], sd).wait()
        plsc.subcore_barrier()

        pltpu.sync_copy(sh_den, den_v)

        def _hist(kk, carry):
            b = kk % 3
            nb = (kk + 1) % 3

            @pl.when(kk >= 2)
            def _():
                pltpu.make_async_copy(ex1.at[pl.ds((kk - 2) * CH, CH)],
                                      sh_c.at[ix2.at[nb]], sd).wait()
            for j in range(8):
                sl = pl.ds(kk * CH + j * 16, 16)
                hx = pk1[sl]
                den16 = plsc.load_gather(den_v, [jnp.right_shift(hx, 5)])
                ex1[sl] = ex1[sl] / (den16 + 1e-16)
                ix2[b, pl.ds(j * 16, 16)] = hx
            pltpu.async_copy(ex1.at[pl.ds(kk * CH, CH)],
                             sh_c.at[ix2.at[b]], sd, add=True)
            return carry
        lax.fori_loop(0, nk, _hist, 0)

        for back in (2, 1):
            @pl.when(nk >= back)
            def _(back=back):
                lb = (nk - back) % 3
                pltpu.make_async_copy(ex1.at[pl.ds((nk - back) * CH, CH)],
                                      sh_c.at[ix2.at[lb]], sd).wait()
        plsc.subcore_barrier()

        pltpu.sync_copy(
            sh_c.at[pl.ds(s * (HALF // 16) * CP, (HALF // 16) * CP)],
            out_h.at[pl.ds(c * HALF * CP + s * (HALF // 16) * CP,
                           (HALF // 16) * CP)])

    return sc_layer0


@functools.lru_cache(maxsize=None)
def _get_sc_layer1():
    @functools.partial(
        pl.kernel,
        out_type=jax.ShapeDtypeStruct((NP, D), _f32),
        mesh=_sc_mesh(),
        compiler_params=pltpu.CompilerParams(needs_layout_passes=False),
        scratch_types=[
            pltpu.VMEM((NP,), _f32),
            pltpu.VMEM((HPAD,), _f32),
            pltpu.VMEM((ECAP,), _i32),
            pltpu.VMEM((HPAD,), _f32),
            pltpu.VMEM((3 * CHR, D), _f32),
            pltpu.VMEM((8, D), _f32),
            pltpu.VMEM((HSTRIPE,), _f32),
            pltpu.VMEM((3, CHR), _i32),
            pltpu.VMEM((3, CHR), _i32),
            pltpu.VMEM((3, CHR), _f32),
            pltpu.VMEM_SHARED((HPAD,), _f32),
            pltpu.VMEM_SHARED((HPAD, D), _f32),
            pltpu.SemaphoreType.DMA,
            pltpu.SemaphoreType.DMA,
            pltpu.SemaphoreType.DMA,
        ],
    )
    def sc_layer1(hh, ash, adh, pkh, out_h,
                  as_v, adh_v, pk1, den_v, rows3, zrow_v, zden_v,
                  ix2, is2, ex2, sh_den, sh_out, sg, ss, sd):
        s = lax.axis_index("s")
        c = lax.axis_index("c")
        base = c * HALF

        pltpu.sync_copy(ash, as_v)
        pltpu.sync_copy(adh.at[pl.ds(base, HALF)], adh_v.at[pl.ds(0, HALF)])
        for j in range(8):
            adh_v[pl.ds(HALF + j * 16, 16)] = jnp.zeros((16,), _f32)
        pltpu.sync_copy(pkh.at[pl.ds(s * ETILE, ETILE)],
                        pk1.at[pl.ds(0, ETILE)])

        def _mx(ref, ng):
            def step(i, m):
                return jnp.maximum(m, ref[pl.ds(i * 16, 16)])
            return _lane_max(lax.fori_loop(0, ng, step,
                                           jnp.full((16,), -1e30, _f32)))
        ngroups = 305 + (1 - c) * 15
        shift = _leaky(_mx(as_v, N // 16) + _mx(adh_v, ngroups))

        def _z1(i, carry):
            zden_v[pl.ds(i * 16, 16)] = jnp.zeros((16,), _f32)
            return carry
        lax.fori_loop(0, HSTRIPE // 16, _z1, 0)

        def _z2(i, carry):
            for j in range(8):
                zrow_v[i, pl.ds(j * 16, 16)] = jnp.zeros((16,), _f32)
            return carry
        lax.fori_loop(0, 8, _z2, 0)
        pltpu.sync_copy(zden_v, sh_den.at[pl.ds(s * HSTRIPE, HSTRIPE)])

        def _z3(t, carry):
            pltpu.sync_copy(zrow_v,
                            sh_out.at[pl.ds(s * HSTRIPE + t * 8, 8)])
            return carry
        lax.fori_loop(0, HSTRIPE // 8, _z3, 0)
        plsc.subcore_barrier()

        def _scan(g, cnt):
            sl = pl.ds(g * 16, 16)
            p16 = pk1[sl]
            ld = jnp.right_shift(p16, 14) - base
            m = (ld >= 0) & (ld < HALF)
            plsc.store_compressed(pk1.at[pl.ds(cnt, 16)],
                                  p16 - base * PK, mask=m)
            return cnt + plsc.all_reduce_population_count(m)[0]
        cnt = lax.fori_loop(0, ETILE // 16, _scan, 0)
        ii = lax.iota(_i32, 16)
        for j in range(CHR // 16):
            idx = cnt + j * 16 + ii
            plsc.store_scatter(pk1, [idx], (HALF + j * 16 + ii) * PK)
        nk = (cnt + CHR - 1) // CHR

        def _prep(kk, b):
            for j in range(CHR // 16):
                slo = pl.ds(j * 16, 16)
                p16 = pk1[pl.ds(kk * CHR + j * 16, 16)]
                ix2[b, slo] = jnp.right_shift(p16, 14)
                is2[b, slo] = p16 & (PK - 1)

        def _exw(b, j):
            slo = pl.ds(j * 16, 16)
            a = plsc.load_gather(as_v, [is2[b, slo]])
            bb = plsc.load_gather(adh_v, [ix2[b, slo]])
            return jnp.exp(_leaky(a + bb) - shift)

        def _den(kk, carry):
            b = kk % 3
            nb = (kk + 1) % 3

            @pl.when(kk >= 2)
            def _():
                pltpu.make_async_copy(ex2.at[nb], sh_den.at[ix2.at[nb]],
                                      sd).wait()
            _prep(kk, b)
            for j in range(CHR // 16):
                ex2[b, pl.ds(j * 16, 16)] = _exw(b, j)
            pltpu.async_copy(ex2.at[b], sh_den.at[ix2.at[b]], sd, add=True)
            return carry
        lax.fori_loop(0, nk, _den, 0)

        for back in (2, 1):
            @pl.when(nk >= back)
            def _(back=back):
                lb = (nk - back) % 3
                pltpu.make_async_copy(ex2.at[lb], sh_den.at[ix2.at[lb]],
                                      sd).wait()
        plsc.subcore_barrier()

        pltpu.sync_copy(sh_den, den_v)

        @pl.when(nk > 0)
        def _():
            _prep(0, 0)
            pltpu.async_copy(hh.at[is2.at[0]], rows3.at[pl.ds(0, CHR)], sg)

        def _rows(kk, carry):
            b = kk % 3
            nb = (kk + 1) % 3

            @pl.when(kk >= 2)
            def _():
                pltpu.make_async_copy(rows3.at[pl.ds(nb * CHR, CHR)],
                                      sh_out.at[ix2.at[nb]], ss).wait()

            @pl.when(kk + 1 < nk)
            def _():
                _prep(kk + 1, nb)
                pltpu.async_copy(hh.at[is2.at[nb]],
                                 rows3.at[pl.ds(nb * CHR, CHR)], sg)

            for j in range(CHR // 16):
                slo = pl.ds(j * 16, 16)
                den16 = plsc.load_gather(den_v, [ix2[b, slo]])
                ex2[b, slo] = _exw(b, j) / (den16 + 1e-16)
            pltpu.make_async_copy(hh.at[is2.at[b]],
                                  rows3.at[pl.ds(b * CHR, CHR)], sg).wait()

            for j in range(CHR // 16):
                coef16 = ex2[b, pl.ds(j * 16, 16)]
                for l in range(16):
                    cc = coef16[l]
                    e = b * CHR + j * 16 + l
                    for jj in range(8):
                        sl2 = pl.ds(jj * 16, 16)
                        rows3[e, sl2] = rows3[e, sl2] * cc
            pltpu.async_copy(rows3.at[pl.ds(b * CHR, CHR)],
                             sh_out.at[ix2.at[b]], ss, add=True)
            return carry
        lax.fori_loop(0, nk, _rows, 0)

        for back in (2, 1):
            @pl.when(nk >= back)
            def _(back=back):
                lb = (nk - back) % 3
                pltpu.make_async_copy(rows3.at[pl.ds(lb * CHR, CHR)],
                                      sh_out.at[ix2.at[lb]], ss).wait()
        plsc.subcore_barrier()

        pltpu.sync_copy(sh_out.at[pl.ds(s * (HALF // 16), HALF // 16)],
                        out_h.at[pl.ds(c * HALF + s * (HALF // 16),
                                       HALF // 16)])

    return sc_layer1


def _tc_prep(ftp, W0, asw, adw):
    def body(ft_r, w_r, as_r, ad_r, t0_r, s_r, d_r):
        t0 = jnp.dot(ft_r[...], w_r[...], preferred_element_type=_f32)
        t0_r[...] = t0
        rid = lax.broadcasted_iota(_i32, (CP, 1), 0)
        sval = jnp.dot(t0, as_r[...], preferred_element_type=_f32)
        dval = jnp.dot(t0, ad_r[...], preferred_element_type=_f32)
        s_r[...] = jnp.where(rid < 21, sval, -1e30)
        d_r[...] = jnp.where(rid < 21, dval, -1e30)

    return pl.pallas_call(
        body,
        out_shape=[
            jax.ShapeDtypeStruct((CP, D), _f32),
            jax.ShapeDtypeStruct((CP, 1), _f32),
            jax.ShapeDtypeStruct((CP, 1), _f32),
        ],
    )(ftp, W0, asw, adw)


_R = 1024


def _tc_mid(cmat, T0p, b0, W1, asw, adw):
    def body(c_r, t0_r, b0_r, w1_r, as_r, ad_r, h_r, s_r, d_r):
        h1 = jnp.maximum(
            jnp.dot(c_r[...], t0_r[...], preferred_element_type=_f32)
            + b0_r[...], 0.0)
        H1 = jnp.dot(h1, w1_r[...], preferred_element_type=_f32)
        h_r[...] = H1
        s_r[...] = jnp.dot(H1, as_r[...], preferred_element_type=_f32)
        d_r[...] = jnp.dot(H1, ad_r[...], preferred_element_type=_f32)

    grid = (NP // _R,)
    return pl.pallas_call(
        body,
        grid=grid,
        in_specs=[
            pl.BlockSpec((_R, CP), lambda i: (i, 0)),
            pl.BlockSpec((CP, D), lambda i: (0, 0)),
            pl.BlockSpec((1, D), lambda i: (0, 0)),
            pl.BlockSpec((D, D), lambda i: (0, 0)),
            pl.BlockSpec((D, 1), lambda i: (0, 0)),
            pl.BlockSpec((D, 1), lambda i: (0, 0)),
        ],
        out_specs=[
            pl.BlockSpec((_R, D), lambda i: (i, 0)),
            pl.BlockSpec((_R, 1), lambda i: (i, 0)),
            pl.BlockSpec((_R, 1), lambda i: (i, 0)),
        ],
        out_shape=[
            jax.ShapeDtypeStruct((NP, D), _f32),
            jax.ShapeDtypeStruct((NP, 1), _f32),
            jax.ShapeDtypeStruct((NP, 1), _f32),
        ],
    )(cmat, T0p, b0, W1, asw, adw)


def _tc_final(o, b1, batchp, fW1, fb1, fW2, fb2):
    def body(o_r, b1_r, bt_r, w1_r, q1_r, w2_r, q2_r, out_r, g_acc):
        i = pl.program_id(0)

        @pl.when(i == 0)
        def _():
            g_acc[...] = jnp.zeros_like(g_acc)

        h2 = jnp.maximum(o_r[...] + b1_r[...], 0.0)
        col = lax.broadcasted_iota(_i32, (_R, B), 1)
        oh = (bt_r[...] == col).astype(_f32)
        g_acc[...] += lax.dot_general(
            oh, h2, (((0,), (0,)), ((), ())), preferred_element_type=_f32)

        @pl.when(i == NP // _R - 1)
        def _():
            g = g_acc[...]
            z = jnp.maximum(
                jnp.dot(g, w1_r[...], preferred_element_type=_f32) + q1_r[...],
                0.0)
            out_r[...] = (jnp.dot(z, w2_r[...], preferred_element_type=_f32)
                          + q2_r[...])

    grid = (NP // _R,)
    return pl.pallas_call(
        body,
        grid=grid,
        in_specs=[
            pl.BlockSpec((_R, D), lambda i: (i, 0)),
            pl.BlockSpec((1, D), lambda i: (0, 0)),
            pl.BlockSpec((_R, 1), lambda i: (i, 0)),
            pl.BlockSpec((D, 2 * D), lambda i: (0, 0)),
            pl.BlockSpec((1, 2 * D), lambda i: (0, 0)),
            pl.BlockSpec((2 * D, T), lambda i: (0, 0)),
            pl.BlockSpec((1, T), lambda i: (0, 0)),
        ],
        out_specs=pl.BlockSpec((B, T), lambda i: (0, 0)),
        out_shape=jax.ShapeDtypeStruct((B, T), _f32),
        scratch_shapes=[pltpu.VMEM((B, B), _f32)],
    )(o, b1, batchp, fW1, fb1, fW2, fb2)


def kernel(x, edge_index, edge_attr, batch, feat_table, W0, att_s0, att_d0, b0,
           W1, att_s1, att_d1, b1, fW1, fb1, fW2, fb2):
    n = x.shape[0]
    e = edge_index.shape[1]

    loop = jnp.arange(n, dtype=_i32)
    npad = EPAD - e - n
    src = jnp.concatenate([edge_index[0].astype(_i32), loop,
                           jnp.zeros((npad,), _i32)])
    dst = jnp.concatenate([edge_index[1].astype(_i32), loop,
                           jnp.full((npad,), NP, _i32)])
    pk = dst * PK + src
    xp = jnp.pad(x.astype(_i32), (0, NP - n))
    ftp = jnp.pad(feat_table, ((0, CP - 21), (0, 0)))

    T0p, s0t, d0t = _tc_prep(ftp, W0, att_s0.reshape(D, 1),
                             att_d0.reshape(D, 1))

    cflat = _get_sc_layer0()(xp, pk, s0t.reshape(CP), d0t.reshape(CP))
    cmat = cflat.reshape(NP, CP)

    H1, as1, ad1 = _tc_mid(cmat, T0p, b0.reshape(1, D), W1,
                           att_s1.reshape(D, 1), att_d1.reshape(D, 1))

    o1 = _get_sc_layer1()(H1, as1.reshape(NP), ad1.reshape(NP), pk)

    batchp = jnp.pad(batch.astype(_i32), (0, NP - n),
                     constant_values=B).reshape(NP, 1)
    return _tc_final(o1, b1.reshape(1, D), batchp, fW1,
                     fb1.reshape(1, 2 * D), fW2, fb2.reshape(1, T))


# speedup vs baseline: 58.9454x; 1.0572x over previous
import functools

import jax
import jax.numpy as jnp
from jax import lax
from jax.experimental import pallas as pl
from jax.experimental.pallas import tpu as pltpu
from jax.experimental.pallas import tpu_sc as plsc

N = 10000
NP = 10240
HALF = NP // 2
HPAD = 5248
D = 128
B = 128
T = 10
CP = 32
CH = 128
CHR = 80
PK = 16384
EPAD = 344064
ETILE = EPAD // 16
ECAP = ETILE + CH
HSTRIPE = HPAD // 16

_f32 = jnp.float32
_i32 = jnp.int32


def _leaky(u):
    return jnp.where(u >= 0, u, 0.2 * u)


_GDN = lax.GatherDimensionNumbers(
    offset_dims=(), collapsed_slice_dims=(0,), start_index_map=(0,))


def _perm(v, idx):
    return lax.gather(v, idx[:, None], _GDN, (1,),
                      mode=lax.GatherScatterMode.PROMISE_IN_BOUNDS)


def _lane_max(v):
    idx = lax.iota(_i32, 16)
    for s2 in (8, 4, 2, 1):
        v = jnp.maximum(v, _perm(v, idx ^ s2))
    return v


def _sc_mesh():
    return plsc.VectorSubcoreMesh(core_axis_name="c", subcore_axis_name="s",
                                  num_cores=2, num_subcores=16)


@functools.lru_cache(maxsize=None)
def _get_sc_layer0():
    @functools.partial(
        pl.kernel,
        out_type=jax.ShapeDtypeStruct((NP * CP,), _f32),
        mesh=_sc_mesh(),
        compiler_params=pltpu.CompilerParams(needs_layout_passes=False),
        scratch_types=[
            pltpu.VMEM((NP,), _i32),
            pltpu.VMEM((CP,), _f32),
            pltpu.VMEM((CP,), _f32),
            pltpu.VMEM((ECAP,), _i32),
            pltpu.VMEM((ECAP,), _f32),
            pltpu.VMEM((HPAD,), _f32),
            pltpu.VMEM((2624,), _f32),
            pltpu.VMEM((3, CH), _i32),
            pltpu.VMEM_SHARED((HPAD,), _f32),
            pltpu.VMEM_SHARED((HPAD * CP,), _f32),
            pltpu.SemaphoreType.DMA,
        ],
    )
    def sc_layer0(xh, pkh, s0h, d0h, out_h,
                  x_v, s0_v, d0_v, pk1, ex1, den_v, zero_v, ix2,
                  sh_den, sh_c, sd):
        s = lax.axis_index("s")
        c = lax.axis_index("c")
        base = c * HALF

        pltpu.sync_copy(xh, x_v)
        pltpu.sync_copy(s0h, s0_v)
        pltpu.sync_copy(d0h, d0_v)
        pltpu.sync_copy(pkh.at[pl.ds(s * ETILE, ETILE)],
                        pk1.at[pl.ds(0, ETILE)])

        ms = _lane_max(jnp.maximum(s0_v[pl.ds(0, 16)], s0_v[pl.ds(16, 16)]))
        md = _lane_max(jnp.maximum(d0_v[pl.ds(0, 16)], d0_v[pl.ds(16, 16)]))
        shift = _leaky(ms + md)

        def _z(i, carry):
            zero_v[pl.ds(i * 16, 16)] = jnp.zeros((16,), _f32)
            return carry
        lax.fori_loop(0, 164, _z, 0)
        pltpu.sync_copy(zero_v.at[pl.ds(0, HSTRIPE)],
                        sh_den.at[pl.ds(s * HSTRIPE, HSTRIPE)])

        def _zc(t, carry):
            pltpu.sync_copy(
                zero_v, sh_c.at[pl.ds((s * 4 + t) * 2624, 2624)])
            return carry
        lax.fori_loop(0, 4, _zc, 0)
        plsc.subcore_barrier()

        def _scan(g, cnt):
            sl = pl.ds(g * 16, 16)
            p16 = pk1[sl]
            s16 = p16 & (PK - 1)
            d16 = jnp.right_shift(p16, 14)
            xs = plsc.load_gather(x_v, [s16])
            xd = plsc.load_gather(x_v, [jnp.minimum(d16, NP - 1)])
            a = plsc.load_gather(s0_v, [xs])
            b = plsc.load_gather(d0_v, [xd])
            ex = jnp.exp(_leaky(a + b) - shift)
            ld = d16 - base
            m = (ld >= 0) & (ld < HALF)
            plsc.store_compressed(pk1.at[pl.ds(cnt, 16)], ld * CP + xs,
                                  mask=m)
            plsc.store_compressed(ex1.at[pl.ds(cnt, 16)], ex, mask=m)
            return cnt + plsc.all_reduce_population_count(m)[0]
        cnt = lax.fori_loop(0, ETILE // 16, _scan, 0)
        ii = lax.iota(_i32, 16)
        for j in range(8):
            idx = cnt + j * 16 + ii
            plsc.store_scatter(pk1, [idx], (HALF + j * 16 + ii) * CP)
            plsc.store_scatter(ex1, [idx], jnp.zeros((16,), _f32))
        nk = (cnt + CH - 1) // CH

        def _den(kk, carry):
            b = kk % 3
            nb = (kk + 1) % 3

            @pl.when(kk >= 2)
            def _():
                pltpu.make_async_copy(ex1.at[pl.ds((kk - 2) * CH, CH)],
                                      sh_den.at[ix2.at[nb]], sd).wait()
            for j in range(8):
                slo = pl.ds(j * 16, 16)
                ix2[b, slo] = jnp.right_shift(
                    pk1[pl.ds(kk * CH + j * 16, 16)], 5)
            pltpu.async_copy(ex1.at[pl.ds(kk * CH, CH)],
                             sh_den.at[ix2.at[b]], sd, add=True)
            return carry
        lax.fori_loop(0, nk, _den, 0)

        for back in (2, 1):
            @pl.when(nk >= back)
            def _(back=back):
                lb = (nk - back) % 3
                pltpu.make_async_copy(ex1.at[pl.ds((nk - back) * CH, CH)],
                                      sh_den.at[ix2.at[lb]], sd).wait()
        plsc.subcore_barrier()

        pltpu.sync_copy(sh_den, den_v)

        def _hist(kk, carry):
            b = kk % 3
            nb = (kk + 1) % 3

            @pl.when(kk >= 2)
            def _():
                pltpu.make_async_copy(ex1.at[pl.ds((kk - 2) * CH, CH)],
                                      sh_c.at[ix2.at[nb]], sd).wait()
            for j in range(8):
                sl = pl.ds(kk * CH + j * 16, 16)
                hx = pk1[sl]
                den16 = plsc.load_gather(den_v, [jnp.right_shift(hx, 5)])
                ex1[sl] = ex1[sl] / (den16 + 1e-16)
                ix2[b, pl.ds(j * 16, 16)] = hx
            pltpu.async_copy(ex1.at[pl.ds(kk * CH, CH)],
                             sh_c.at[ix2.at[b]], sd, add=True)
            return carry
        lax.fori_loop(0, nk, _hist, 0)

        for back in (2, 1):
            @pl.when(nk >= back)
            def _(back=back):
                lb = (nk - back) % 3
                pltpu.make_async_copy(ex1.at[pl.ds((nk - back) * CH, CH)],
                                      sh_c.at[ix2.at[lb]], sd).wait()
        plsc.subcore_barrier()

        pltpu.sync_copy(
            sh_c.at[pl.ds(s * (HALF // 16) * CP, (HALF // 16) * CP)],
            out_h.at[pl.ds(c * HALF * CP + s * (HALF // 16) * CP,
                           (HALF // 16) * CP)])

    return sc_layer0


@functools.lru_cache(maxsize=None)
def _get_sc_layer1():
    @functools.partial(
        pl.kernel,
        out_type=jax.ShapeDtypeStruct((NP, D), _f32),
        mesh=_sc_mesh(),
        compiler_params=pltpu.CompilerParams(needs_layout_passes=False),
        scratch_types=[
            pltpu.VMEM((NP,), _f32),
            pltpu.VMEM((HPAD,), _f32),
            pltpu.VMEM((ECAP,), _i32),
            pltpu.VMEM((HPAD,), _f32),
            pltpu.VMEM((4 * CHR, D), _f32),
            pltpu.VMEM((8, D), _f32),
            pltpu.VMEM((HSTRIPE,), _f32),
            pltpu.VMEM((4, CHR), _i32),
            pltpu.VMEM((4, CHR), _i32),
            pltpu.VMEM((4, CHR), _f32),
            pltpu.VMEM_SHARED((HPAD,), _f32),
            pltpu.VMEM_SHARED((HPAD, D), _f32),
            pltpu.SemaphoreType.DMA,
            pltpu.SemaphoreType.DMA,
            pltpu.SemaphoreType.DMA,
        ],
    )
    def sc_layer1(hh, ash, adh, pkh, out_h,
                  as_v, adh_v, pk1, den_v, rows4, zrow_v, zden_v,
                  ix2, is2, ex2, sh_den, sh_out, sg, ss, sd):
        s = lax.axis_index("s")
        c = lax.axis_index("c")
        base = c * HALF

        pltpu.sync_copy(ash, as_v)
        pltpu.sync_copy(adh.at[pl.ds(base, HALF)], adh_v.at[pl.ds(0, HALF)])
        for j in range(8):
            adh_v[pl.ds(HALF + j * 16, 16)] = jnp.zeros((16,), _f32)
        pltpu.sync_copy(pkh.at[pl.ds(s * ETILE, ETILE)],
                        pk1.at[pl.ds(0, ETILE)])

        def _mx(ref, ng):
            def step(i, m):
                return jnp.maximum(m, ref[pl.ds(i * 16, 16)])
            return _lane_max(lax.fori_loop(0, ng, step,
                                           jnp.full((16,), -1e30, _f32)))
        ngroups = 305 + (1 - c) * 15
        shift = _leaky(_mx(as_v, N // 16) + _mx(adh_v, ngroups))

        def _z1(i, carry):
            zden_v[pl.ds(i * 16, 16)] = jnp.zeros((16,), _f32)
            return carry
        lax.fori_loop(0, HSTRIPE // 16, _z1, 0)

        def _z2(i, carry):
            for j in range(8):
                zrow_v[i, pl.ds(j * 16, 16)] = jnp.zeros((16,), _f32)
            return carry
        lax.fori_loop(0, 8, _z2, 0)
        pltpu.sync_copy(zden_v, sh_den.at[pl.ds(s * HSTRIPE, HSTRIPE)])

        def _z3(t, carry):
            pltpu.sync_copy(zrow_v,
                            sh_out.at[pl.ds(s * HSTRIPE + t * 8, 8)])
            return carry
        lax.fori_loop(0, HSTRIPE // 8, _z3, 0)
        plsc.subcore_barrier()

        def _scan(g, cnt):
            sl = pl.ds(g * 16, 16)
            p16 = pk1[sl]
            ld = jnp.right_shift(p16, 14) - base
            m = (ld >= 0) & (ld < HALF)
            plsc.store_compressed(pk1.at[pl.ds(cnt, 16)],
                                  p16 - base * PK, mask=m)
            return cnt + plsc.all_reduce_population_count(m)[0]
        cnt = lax.fori_loop(0, ETILE // 16, _scan, 0)
        ii = lax.iota(_i32, 16)
        for j in range(CHR // 16):
            idx = cnt + j * 16 + ii
            plsc.store_scatter(pk1, [idx], (HALF + j * 16 + ii) * PK)
        nk = (cnt + CHR - 1) // CHR

        def _prep(kk, b):
            for j in range(CHR // 16):
                slo = pl.ds(j * 16, 16)
                p16 = pk1[pl.ds(kk * CHR + j * 16, 16)]
                ix2[b, slo] = jnp.right_shift(p16, 14)
                is2[b, slo] = p16 & (PK - 1)

        def _exw(b, j):
            slo = pl.ds(j * 16, 16)
            a = plsc.load_gather(as_v, [is2[b, slo]])
            bb = plsc.load_gather(adh_v, [ix2[b, slo]])
            return jnp.exp(_leaky(a + bb) - shift)

        def _den(kk, carry):
            b = kk % 4

            @pl.when(kk >= 4)
            def _():
                pltpu.make_async_copy(ex2.at[b], sh_den.at[ix2.at[b]],
                                      sd).wait()
            _prep(kk, b)
            for j in range(CHR // 16):
                ex2[b, pl.ds(j * 16, 16)] = _exw(b, j)
            pltpu.async_copy(ex2.at[b], sh_den.at[ix2.at[b]], sd, add=True)
            return carry
        lax.fori_loop(0, nk, _den, 0)

        for back in (4, 3, 2, 1):
            @pl.when(nk >= back)
            def _(back=back):
                lb = (nk - back) % 4
                pltpu.make_async_copy(ex2.at[lb], sh_den.at[ix2.at[lb]],
                                      sd).wait()
        plsc.subcore_barrier()

        pltpu.sync_copy(sh_den, den_v)

        @pl.when(nk > 0)
        def _():
            _prep(0, 0)
            pltpu.async_copy(hh.at[is2.at[0]], rows4.at[pl.ds(0, CHR)], sg)

        @pl.when(nk > 1)
        def _():
            _prep(1, 1)
            pltpu.async_copy(hh.at[is2.at[1]], rows4.at[pl.ds(CHR, CHR)], sg)

        def _rows(kk, carry):
            b = kk % 4
            pb = (kk + 2) % 4

            @pl.when(kk >= 2)
            def _():
                pltpu.make_async_copy(rows4.at[pl.ds(pb * CHR, CHR)],
                                      sh_out.at[ix2.at[pb]], ss).wait()

            @pl.when(kk + 2 < nk)
            def _():
                _prep(kk + 2, pb)
                pltpu.async_copy(hh.at[is2.at[pb]],
                                 rows4.at[pl.ds(pb * CHR, CHR)], sg)

            for j in range(CHR // 16):
                slo = pl.ds(j * 16, 16)
                den16 = plsc.load_gather(den_v, [ix2[b, slo]])
                ex2[b, slo] = _exw(b, j) / (den16 + 1e-16)
            pltpu.make_async_copy(hh.at[is2.at[b]],
                                  rows4.at[pl.ds(b * CHR, CHR)], sg).wait()

            for j in range(CHR // 16):
                coef16 = ex2[b, pl.ds(j * 16, 16)]
                for l in range(16):
                    cc = coef16[l]
                    e = b * CHR + j * 16 + l
                    for jj in range(8):
                        sl2 = pl.ds(jj * 16, 16)
                        rows4[e, sl2] = rows4[e, sl2] * cc
            pltpu.async_copy(rows4.at[pl.ds(b * CHR, CHR)],
                             sh_out.at[ix2.at[b]], ss, add=True)
            return carry
        lax.fori_loop(0, nk, _rows, 0)

        for back in (2, 1):
            @pl.when(nk >= back)
            def _(back=back):
                lb = (nk - back) % 4
                pltpu.make_async_copy(rows4.at[pl.ds(lb * CHR, CHR)],
                                      sh_out.at[ix2.at[lb]], ss).wait()
        plsc.subcore_barrier()

        pltpu.sync_copy(sh_out.at[pl.ds(s * (HALF // 16), HALF // 16)],
                        out_h.at[pl.ds(c * HALF + s * (HALF // 16),
                                       HALF // 16)])

    return sc_layer1


def _tc_prep(ftp, W0, asw, adw):
    def body(ft_r, w_r, as_r, ad_r, t0_r, s_r, d_r):
        t0 = jnp.dot(ft_r[...], w_r[...], preferred_element_type=_f32)
        t0_r[...] = t0
        rid = lax.broadcasted_iota(_i32, (CP, 1), 0)
        sval = jnp.dot(t0, as_r[...], preferred_element_type=_f32)
        dval = jnp.dot(t0, ad_r[...], preferred_element_type=_f32)
        s_r[...] = jnp.where(rid < 21, sval, -1e30)
        d_r[...] = jnp.where(rid < 21, dval, -1e30)

    return pl.pallas_call(
        body,
        out_shape=[
            jax.ShapeDtypeStruct((CP, D), _f32),
            jax.ShapeDtypeStruct((CP, 1), _f32),
            jax.ShapeDtypeStruct((CP, 1), _f32),
        ],
    )(ftp, W0, asw, adw)


_R = 1024


def _tc_mid(cmat, T0p, b0, W1, asw, adw):
    def body(c_r, t0_r, b0_r, w1_r, as_r, ad_r, h_r, s_r, d_r):
        h1 = jnp.maximum(
            jnp.dot(c_r[...], t0_r[...], preferred_element_type=_f32)
            + b0_r[...], 0.0)
        H1 = jnp.dot(h1, w1_r[...], preferred_element_type=_f32)
        h_r[...] = H1
        s_r[...] = jnp.dot(H1, as_r[...], preferred_element_type=_f32)
        d_r[...] = jnp.dot(H1, ad_r[...], preferred_element_type=_f32)

    grid = (NP // _R,)
    return pl.pallas_call(
        body,
        grid=grid,
        in_specs=[
            pl.BlockSpec((_R, CP), lambda i: (i, 0)),
            pl.BlockSpec((CP, D), lambda i: (0, 0)),
            pl.BlockSpec((1, D), lambda i: (0, 0)),
            pl.BlockSpec((D, D), lambda i: (0, 0)),
            pl.BlockSpec((D, 1), lambda i: (0, 0)),
            pl.BlockSpec((D, 1), lambda i: (0, 0)),
        ],
        out_specs=[
            pl.BlockSpec((_R, D), lambda i: (i, 0)),
            pl.BlockSpec((_R, 1), lambda i: (i, 0)),
            pl.BlockSpec((_R, 1), lambda i: (i, 0)),
        ],
        out_shape=[
            jax.ShapeDtypeStruct((NP, D), _f32),
            jax.ShapeDtypeStruct((NP, 1), _f32),
            jax.ShapeDtypeStruct((NP, 1), _f32),
        ],
    )(cmat, T0p, b0, W1, asw, adw)


def _tc_final(o, b1, batchp, fW1, fb1, fW2, fb2):
    def body(o_r, b1_r, bt_r, w1_r, q1_r, w2_r, q2_r, out_r, g_acc):
        i = pl.program_id(0)

        @pl.when(i == 0)
        def _():
            g_acc[...] = jnp.zeros_like(g_acc)

        h2 = jnp.maximum(o_r[...] + b1_r[...], 0.0)
        col = lax.broadcasted_iota(_i32, (_R, B), 1)
        oh = (bt_r[...] == col).astype(_f32)
        g_acc[...] += lax.dot_general(
            oh, h2, (((0,), (0,)), ((), ())), preferred_element_type=_f32)

        @pl.when(i == NP // _R - 1)
        def _():
            g = g_acc[...]
            z = jnp.maximum(
                jnp.dot(g, w1_r[...], preferred_element_type=_f32) + q1_r[...],
                0.0)
            out_r[...] = (jnp.dot(z, w2_r[...], preferred_element_type=_f32)
                          + q2_r[...])

    grid = (NP // _R,)
    return pl.pallas_call(
        body,
        grid=grid,
        in_specs=[
            pl.BlockSpec((_R, D), lambda i: (i, 0)),
            pl.BlockSpec((1, D), lambda i: (0, 0)),
            pl.BlockSpec((_R, 1), lambda i: (i, 0)),
            pl.BlockSpec((D, 2 * D), lambda i: (0, 0)),
            pl.BlockSpec((1, 2 * D), lambda i: (0, 0)),
            pl.BlockSpec((2 * D, T), lambda i: (0, 0)),
            pl.BlockSpec((1, T), lambda i: (0, 0)),
        ],
        out_specs=pl.BlockSpec((B, T), lambda i: (0, 0)),
        out_shape=jax.ShapeDtypeStruct((B, T), _f32),
        scratch_shapes=[pltpu.VMEM((B, B), _f32)],
    )(o, b1, batchp, fW1, fb1, fW2, fb2)


def kernel(x, edge_index, edge_attr, batch, feat_table, W0, att_s0, att_d0, b0,
           W1, att_s1, att_d1, b1, fW1, fb1, fW2, fb2):
    n = x.shape[0]
    e = edge_index.shape[1]

    loop = jnp.arange(n, dtype=_i32)
    npad = EPAD - e - n
    src = jnp.concatenate([edge_index[0].astype(_i32), loop,
                           jnp.zeros((npad,), _i32)])
    dst = jnp.concatenate([edge_index[1].astype(_i32), loop,
                           jnp.full((npad,), NP, _i32)])
    pk = dst * PK + src
    xp = jnp.pad(x.astype(_i32), (0, NP - n))
    ftp = jnp.pad(feat_table, ((0, CP - 21), (0, 0)))

    T0p, s0t, d0t = _tc_prep(ftp, W0, att_s0.reshape(D, 1),
                             att_d0.reshape(D, 1))

    cflat = _get_sc_layer0()(xp, pk, s0t.reshape(CP), d0t.reshape(CP))
    cmat = cflat.reshape(NP, CP)

    H1, as1, ad1 = _tc_mid(cmat, T0p, b0.reshape(1, D), W1,
                           att_s1.reshape(D, 1), att_d1.reshape(D, 1))

    o1 = _get_sc_layer1()(H1, as1.reshape(NP), ad1.reshape(NP), pk)

    batchp = jnp.pad(batch.astype(_i32), (0, NP - n),
                     constant_values=B).reshape(NP, 1)
    return _tc_final(o1, b1.reshape(1, D), batchp, fW1,
                     fb1.reshape(1, 2 * D), fW2, fb2.reshape(1, T))


# speedup vs baseline: 59.6738x; 1.0124x over previous
import functools

import jax
import jax.numpy as jnp
from jax import lax
from jax.experimental import pallas as pl
from jax.experimental.pallas import tpu as pltpu
from jax.experimental.pallas import tpu_sc as plsc

N = 10000
NP = 10240
HALF = NP // 2
HPAD = 5248
D = 128
B = 128
T = 10
CP = 32
CH = 128
CHR = 80
PK = 16384
EPAD = 344064
ETILE = EPAD // 16
ECAP = ETILE + CH
HSTRIPE = HPAD // 16

_f32 = jnp.float32
_i32 = jnp.int32


def _leaky(u):
    return jnp.where(u >= 0, u, 0.2 * u)


_GDN = lax.GatherDimensionNumbers(
    offset_dims=(), collapsed_slice_dims=(0,), start_index_map=(0,))


def _perm(v, idx):
    return lax.gather(v, idx[:, None], _GDN, (1,),
                      mode=lax.GatherScatterMode.PROMISE_IN_BOUNDS)


def _lane_max(v):
    idx = lax.iota(_i32, 16)
    for s2 in (8, 4, 2, 1):
        v = jnp.maximum(v, _perm(v, idx ^ s2))
    return v


def _sc_mesh():
    return plsc.VectorSubcoreMesh(core_axis_name="c", subcore_axis_name="s",
                                  num_cores=2, num_subcores=16)


@functools.lru_cache(maxsize=None)
def _get_sc_layer0():
    @functools.partial(
        pl.kernel,
        out_type=jax.ShapeDtypeStruct((NP * CP,), _f32),
        mesh=_sc_mesh(),
        compiler_params=pltpu.CompilerParams(needs_layout_passes=False),
        scratch_types=[
            pltpu.VMEM((NP,), _i32),
            pltpu.VMEM((CP,), _f32),
            pltpu.VMEM((CP,), _f32),
            pltpu.VMEM((ECAP,), _i32),
            pltpu.VMEM((ECAP,), _f32),
            pltpu.VMEM((HPAD,), _f32),
            pltpu.VMEM((2624,), _f32),
            pltpu.VMEM((3, CH), _i32),
            pltpu.VMEM_SHARED((HPAD,), _f32),
            pltpu.VMEM_SHARED((HPAD * CP,), _f32),
            pltpu.SemaphoreType.DMA,
            pltpu.SemaphoreType.DMA,
        ],
    )
    def sc_layer0(xh, pkh, s0h, d0h, out_h,
                  x_v, s0_v, d0_v, pk1, ex1, den_v, zero_v, ix2,
                  sh_den, sh_c, sd, sz):
        s = lax.axis_index("s")
        c = lax.axis_index("c")
        base = c * HALF

        pltpu.sync_copy(xh, x_v)
        pltpu.sync_copy(s0h, s0_v)
        pltpu.sync_copy(d0h, d0_v)
        pltpu.sync_copy(pkh.at[pl.ds(s * ETILE, ETILE)],
                        pk1.at[pl.ds(0, ETILE)])

        ms = _lane_max(jnp.maximum(s0_v[pl.ds(0, 16)], s0_v[pl.ds(16, 16)]))
        md = _lane_max(jnp.maximum(d0_v[pl.ds(0, 16)], d0_v[pl.ds(16, 16)]))
        shift = _leaky(ms + md)

        def _z(i, carry):
            zero_v[pl.ds(i * 16, 16)] = jnp.zeros((16,), _f32)
            return carry
        lax.fori_loop(0, 164, _z, 0)
        pltpu.async_copy(zero_v.at[pl.ds(0, HSTRIPE)],
                         sh_den.at[pl.ds(s * HSTRIPE, HSTRIPE)], sz)
        for t in range(4):
            pltpu.async_copy(
                zero_v, sh_c.at[pl.ds((s * 4 + t) * 2624, 2624)], sz)

        def _scan(g, cnt):
            sl = pl.ds(g * 16, 16)
            p16 = pk1[sl]
            s16 = p16 & (PK - 1)
            d16 = jnp.right_shift(p16, 14)
            xs = plsc.load_gather(x_v, [s16])
            xd = plsc.load_gather(x_v, [jnp.minimum(d16, NP - 1)])
            a = plsc.load_gather(s0_v, [xs])
            b = plsc.load_gather(d0_v, [xd])
            ex = jnp.exp(_leaky(a + b) - shift)
            ld = d16 - base
            m = (ld >= 0) & (ld < HALF)
            plsc.store_compressed(pk1.at[pl.ds(cnt, 16)], ld * CP + xs,
                                  mask=m)
            plsc.store_compressed(ex1.at[pl.ds(cnt, 16)], ex, mask=m)
            return cnt + plsc.all_reduce_population_count(m)[0]
        cnt = lax.fori_loop(0, ETILE // 16, _scan, 0)
        ii = lax.iota(_i32, 16)
        for j in range(8):
            idx = cnt + j * 16 + ii
            plsc.store_scatter(pk1, [idx], (HALF + j * 16 + ii) * CP)
            plsc.store_scatter(ex1, [idx], jnp.zeros((16,), _f32))
        nk = (cnt + CH - 1) // CH

        pltpu.make_async_copy(zero_v.at[pl.ds(0, HSTRIPE)],
                              sh_den.at[pl.ds(s * HSTRIPE, HSTRIPE)],
                              sz).wait()
        for t in range(4):
            pltpu.make_async_copy(
                zero_v, sh_c.at[pl.ds((s * 4 + t) * 2624, 2624)], sz).wait()
        plsc.subcore_barrier()

        def _den(kk, carry):
            b = kk % 3
            nb = (kk + 1) % 3

            @pl.when(kk >= 2)
            def _():
                pltpu.make_async_copy(ex1.at[pl.ds((kk - 2) * CH, CH)],
                                      sh_den.at[ix2.at[nb]], sd).wait()
            for j in range(8):
                slo = pl.ds(j * 16, 16)
                ix2[b, slo] = jnp.right_shift(
                    pk1[pl.ds(kk * CH + j * 16, 16)], 5)
            pltpu.async_copy(ex1.at[pl.ds(kk * CH, CH)],
                             sh_den.at[ix2.at[b]], sd, add=True)
            return carry
        lax.fori_loop(0, nk, _den, 0)

        for back in (2, 1):
            @pl.when(nk >= back)
            def _(back=back):
                lb = (nk - back) % 3
                pltpu.make_async_copy(ex1.at[pl.ds((nk - back) * CH, CH)],
                                      sh_den.at[ix2.at[lb]], sd).wait()
        plsc.subcore_barrier()

        pltpu.sync_copy(sh_den, den_v)

        def _hist(kk, carry):
            b = kk % 3
            nb = (kk + 1) % 3

            @pl.when(kk >= 2)
            def _():
                pltpu.make_async_copy(ex1.at[pl.ds((kk - 2) * CH, CH)],
                                      sh_c.at[ix2.at[nb]], sd).wait()
            for j in range(8):
                sl = pl.ds(kk * CH + j * 16, 16)
                hx = pk1[sl]
                den16 = plsc.load_gather(den_v, [jnp.right_shift(hx, 5)])
                ex1[sl] = ex1[sl] / (den16 + 1e-16)
                ix2[b, pl.ds(j * 16, 16)] = hx
            pltpu.async_copy(ex1.at[pl.ds(kk * CH, CH)],
                             sh_c.at[ix2.at[b]], sd, add=True)
            return carry
        lax.fori_loop(0, nk, _hist, 0)

        for back in (2, 1):
            @pl.when(nk >= back)
            def _(back=back):
                lb = (nk - back) % 3
                pltpu.make_async_copy(ex1.at[pl.ds((nk - back) * CH, CH)],
                                      sh_c.at[ix2.at[lb]], sd).wait()
        plsc.subcore_barrier()

        pltpu.sync_copy(
            sh_c.at[pl.ds(s * (HALF // 16) * CP, (HALF // 16) * CP)],
            out_h.at[pl.ds(c * HALF * CP + s * (HALF // 16) * CP,
                           (HALF // 16) * CP)])

    return sc_layer0


@functools.lru_cache(maxsize=None)
def _get_sc_layer1():
    @functools.partial(
        pl.kernel,
        out_type=jax.ShapeDtypeStruct((NP, D), _f32),
        mesh=_sc_mesh(),
        compiler_params=pltpu.CompilerParams(needs_layout_passes=False),
        scratch_types=[
            pltpu.VMEM((NP,), _f32),
            pltpu.VMEM((HPAD,), _f32),
            pltpu.VMEM((ECAP,), _i32),
            pltpu.VMEM((HPAD,), _f32),
            pltpu.VMEM((4 * CHR, D), _f32),
            pltpu.VMEM((8, D), _f32),
            pltpu.VMEM((HSTRIPE,), _f32),
            pltpu.VMEM((4, CHR), _i32),
            pltpu.VMEM((4, CHR), _i32),
            pltpu.VMEM((4, CHR), _f32),
            pltpu.VMEM_SHARED((HPAD,), _f32),
            pltpu.VMEM_SHARED((HPAD, D), _f32),
            pltpu.SemaphoreType.DMA,
            pltpu.SemaphoreType.DMA,
            pltpu.SemaphoreType.DMA,
            pltpu.SemaphoreType.DMA,
        ],
    )
    def sc_layer1(hh, ash, adh, pkh, out_h,
                  as_v, adh_v, pk1, den_v, rows4, zrow_v, zden_v,
                  ix2, is2, ex2, sh_den, sh_out, sg, ss, sd, sz):
        s = lax.axis_index("s")
        c = lax.axis_index("c")
        base = c * HALF

        pltpu.sync_copy(ash, as_v)
        pltpu.sync_copy(adh.at[pl.ds(base, HALF)], adh_v.at[pl.ds(0, HALF)])
        for j in range(8):
            adh_v[pl.ds(HALF + j * 16, 16)] = jnp.zeros((16,), _f32)
        pltpu.sync_copy(pkh.at[pl.ds(s * ETILE, ETILE)],
                        pk1.at[pl.ds(0, ETILE)])

        def _mx(ref, ng):
            def step(i, m):
                return jnp.maximum(m, ref[pl.ds(i * 16, 16)])
            return _lane_max(lax.fori_loop(0, ng, step,
                                           jnp.full((16,), -1e30, _f32)))
        ngroups = 305 + (1 - c) * 15
        shift = _leaky(_mx(as_v, N // 16) + _mx(adh_v, ngroups))

        def _z1(i, carry):
            zden_v[pl.ds(i * 16, 16)] = jnp.zeros((16,), _f32)
            return carry
        lax.fori_loop(0, HSTRIPE // 16, _z1, 0)

        def _z2(i, carry):
            for j in range(8):
                zrow_v[i, pl.ds(j * 16, 16)] = jnp.zeros((16,), _f32)
            return carry
        lax.fori_loop(0, 8, _z2, 0)
        pltpu.async_copy(zden_v, sh_den.at[pl.ds(s * HSTRIPE, HSTRIPE)], sz)

        def _z3(t, carry):
            pltpu.async_copy(zrow_v,
                             sh_out.at[pl.ds(s * HSTRIPE + t * 8, 8)], sz)
            return carry
        lax.fori_loop(0, HSTRIPE // 8, _z3, 0)

        def _scan(g, cnt):
            sl = pl.ds(g * 16, 16)
            p16 = pk1[sl]
            ld = jnp.right_shift(p16, 14) - base
            m = (ld >= 0) & (ld < HALF)
            plsc.store_compressed(pk1.at[pl.ds(cnt, 16)],
                                  p16 - base * PK, mask=m)
            return cnt + plsc.all_reduce_population_count(m)[0]
        cnt = lax.fori_loop(0, ETILE // 16, _scan, 0)
        ii = lax.iota(_i32, 16)
        for j in range(CHR // 16):
            idx = cnt + j * 16 + ii
            plsc.store_scatter(pk1, [idx], (HALF + j * 16 + ii) * PK)
        nk = (cnt + CHR - 1) // CHR

        pltpu.make_async_copy(zden_v, sh_den.at[pl.ds(s * HSTRIPE, HSTRIPE)],
                              sz).wait()

        def _z3w(t, carry):
            pltpu.make_async_copy(zrow_v,
                                  sh_out.at[pl.ds(s * HSTRIPE + t * 8, 8)],
                                  sz).wait()
            return carry
        lax.fori_loop(0, HSTRIPE // 8, _z3w, 0)
        plsc.subcore_barrier()

        def _prep(kk, b):
            for j in range(CHR // 16):
                slo = pl.ds(j * 16, 16)
                p16 = pk1[pl.ds(kk * CHR + j * 16, 16)]
                ix2[b, slo] = jnp.right_shift(p16, 14)
                is2[b, slo] = p16 & (PK - 1)

        def _exw(b, j):
            slo = pl.ds(j * 16, 16)
            a = plsc.load_gather(as_v, [is2[b, slo]])
            bb = plsc.load_gather(adh_v, [ix2[b, slo]])
            return jnp.exp(_leaky(a + bb) - shift)

        def _den(kk, carry):
            b = kk % 4

            @pl.when(kk >= 4)
            def _():
                pltpu.make_async_copy(ex2.at[b], sh_den.at[ix2.at[b]],
                                      sd).wait()
            _prep(kk, b)
            for j in range(CHR // 16):
                ex2[b, pl.ds(j * 16, 16)] = _exw(b, j)
            pltpu.async_copy(ex2.at[b], sh_den.at[ix2.at[b]], sd, add=True)
            return carry
        lax.fori_loop(0, nk, _den, 0)

        for back in (4, 3, 2, 1):
            @pl.when(nk >= back)
            def _(back=back):
                lb = (nk - back) % 4
                pltpu.make_async_copy(ex2.at[lb], sh_den.at[ix2.at[lb]],
                                      sd).wait()
        plsc.subcore_barrier()

        pltpu.sync_copy(sh_den, den_v)

        @pl.when(nk > 0)
        def _():
            _prep(0, 0)
            pltpu.async_copy(hh.at[is2.at[0]], rows4.at[pl.ds(0, CHR)], sg)

        @pl.when(nk > 1)
        def _():
            _prep(1, 1)
            pltpu.async_copy(hh.at[is2.at[1]], rows4.at[pl.ds(CHR, CHR)], sg)

        def _rows(kk, carry):
            b = kk % 4
            pb = (kk + 2) % 4

            @pl.when(kk >= 2)
            def _():
                pltpu.make_async_copy(rows4.at[pl.ds(pb * CHR, CHR)],
                                      sh_out.at[ix2.at[pb]], ss).wait()

            @pl.when(kk + 2 < nk)
            def _():
                _prep(kk + 2, pb)
                pltpu.async_copy(hh.at[is2.at[pb]],
                                 rows4.at[pl.ds(pb * CHR, CHR)], sg)

            for j in range(CHR // 16):
                slo = pl.ds(j * 16, 16)
                den16 = plsc.load_gather(den_v, [ix2[b, slo]])
                ex2[b, slo] = _exw(b, j) / (den16 + 1e-16)
            pltpu.make_async_copy(hh.at[is2.at[b]],
                                  rows4.at[pl.ds(b * CHR, CHR)], sg).wait()

            for j in range(CHR // 16):
                coef16 = ex2[b, pl.ds(j * 16, 16)]
                for l in range(16):
                    cc = coef16[l]
                    e = b * CHR + j * 16 + l
                    for jj in range(8):
                        sl2 = pl.ds(jj * 16, 16)
                        rows4[e, sl2] = rows4[e, sl2] * cc
            pltpu.async_copy(rows4.at[pl.ds(b * CHR, CHR)],
                             sh_out.at[ix2.at[b]], ss, add=True)
            return carry
        lax.fori_loop(0, nk, _rows, 0)

        for back in (2, 1):
            @pl.when(nk >= back)
            def _(back=back):
                lb = (nk - back) % 4
                pltpu.make_async_copy(rows4.at[pl.ds(lb * CHR, CHR)],
                                      sh_out.at[ix2.at[lb]], ss).wait()
        plsc.subcore_barrier()

        pltpu.sync_copy(sh_out.at[pl.ds(s * (HALF // 16), HALF // 16)],
                        out_h.at[pl.ds(c * HALF + s * (HALF // 16),
                                       HALF // 16)])

    return sc_layer1


def _tc_prep(ftp, W0, asw, adw):
    def body(ft_r, w_r, as_r, ad_r, t0_r, s_r, d_r):
        t0 = jnp.dot(ft_r[...], w_r[...], preferred_element_type=_f32)
        t0_r[...] = t0
        rid = lax.broadcasted_iota(_i32, (CP, 1), 0)
        sval = jnp.dot(t0, as_r[...], preferred_element_type=_f32)
        dval = jnp.dot(t0, ad_r[...], preferred_element_type=_f32)
        s_r[...] = jnp.where(rid < 21, sval, -1e30)
        d_r[...] = jnp.where(rid < 21, dval, -1e30)

    return pl.pallas_call(
        body,
        out_shape=[
            jax.ShapeDtypeStruct((CP, D), _f32),
            jax.ShapeDtypeStruct((CP, 1), _f32),
            jax.ShapeDtypeStruct((CP, 1), _f32),
        ],
    )(ftp, W0, asw, adw)


_R = 1024


def _tc_mid(cmat, T0p, b0, W1, asw, adw):
    def body(c_r, t0_r, b0_r, w1_r, as_r, ad_r, h_r, s_r, d_r):
        h1 = jnp.maximum(
            jnp.dot(c_r[...], t0_r[...], preferred_element_type=_f32)
            + b0_r[...], 0.0)
        H1 = jnp.dot(h1, w1_r[...], preferred_element_type=_f32)
        h_r[...] = H1
        s_r[...] = jnp.dot(H1, as_r[...], preferred_element_type=_f32)
        d_r[...] = jnp.dot(H1, ad_r[...], preferred_element_type=_f32)

    grid = (NP // _R,)
    return pl.pallas_call(
        body,
        grid=grid,
        in_specs=[
            pl.BlockSpec((_R, CP), lambda i: (i, 0)),
            pl.BlockSpec((CP, D), lambda i: (0, 0)),
            pl.BlockSpec((1, D), lambda i: (0, 0)),
            pl.BlockSpec((D, D), lambda i: (0, 0)),
            pl.BlockSpec((D, 1), lambda i: (0, 0)),
            pl.BlockSpec((D, 1), lambda i: (0, 0)),
        ],
        out_specs=[
            pl.BlockSpec((_R, D), lambda i: (i, 0)),
            pl.BlockSpec((_R, 1), lambda i: (i, 0)),
            pl.BlockSpec((_R, 1), lambda i: (i, 0)),
        ],
        out_shape=[
            jax.ShapeDtypeStruct((NP, D), _f32),
            jax.ShapeDtypeStruct((NP, 1), _f32),
            jax.ShapeDtypeStruct((NP, 1), _f32),
        ],
    )(cmat, T0p, b0, W1, asw, adw)


def _tc_final(o, b1, batchp, fW1, fb1, fW2, fb2):
    def body(o_r, b1_r, bt_r, w1_r, q1_r, w2_r, q2_r, out_r, g_acc):
        i = pl.program_id(0)

        @pl.when(i == 0)
        def _():
            g_acc[...] = jnp.zeros_like(g_acc)

        h2 = jnp.maximum(o_r[...] + b1_r[...], 0.0)
        col = lax.broadcasted_iota(_i32, (_R, B), 1)
        oh = (bt_r[...] == col).astype(_f32)
        g_acc[...] += lax.dot_general(
            oh, h2, (((0,), (0,)), ((), ())), preferred_element_type=_f32)

        @pl.when(i == NP // _R - 1)
        def _():
            g = g_acc[...]
            z = jnp.maximum(
                jnp.dot(g, w1_r[...], preferred_element_type=_f32) + q1_r[...],
                0.0)
            out_r[...] = (jnp.dot(z, w2_r[...], preferred_element_type=_f32)
                          + q2_r[...])

    grid = (NP // _R,)
    return pl.pallas_call(
        body,
        grid=grid,
        in_specs=[
            pl.BlockSpec((_R, D), lambda i: (i, 0)),
            pl.BlockSpec((1, D), lambda i: (0, 0)),
            pl.BlockSpec((_R, 1), lambda i: (i, 0)),
            pl.BlockSpec((D, 2 * D), lambda i: (0, 0)),
            pl.BlockSpec((1, 2 * D), lambda i: (0, 0)),
            pl.BlockSpec((2 * D, T), lambda i: (0, 0)),
            pl.BlockSpec((1, T), lambda i: (0, 0)),
        ],
        out_specs=pl.BlockSpec((B, T), lambda i: (0, 0)),
        out_shape=jax.ShapeDtypeStruct((B, T), _f32),
        scratch_shapes=[pltpu.VMEM((B, B), _f32)],
    )(o, b1, batchp, fW1, fb1, fW2, fb2)


def kernel(x, edge_index, edge_attr, batch, feat_table, W0, att_s0, att_d0, b0,
           W1, att_s1, att_d1, b1, fW1, fb1, fW2, fb2):
    n = x.shape[0]
    e = edge_index.shape[1]

    loop = jnp.arange(n, dtype=_i32)
    npad = EPAD - e - n
    src = jnp.concatenate([edge_index[0].astype(_i32), loop,
                           jnp.zeros((npad,), _i32)])
    dst = jnp.concatenate([edge_index[1].astype(_i32), loop,
                           jnp.full((npad,), NP, _i32)])
    pk = dst * PK + src
    xp = jnp.pad(x.astype(_i32), (0, NP - n))
    ftp = jnp.pad(feat_table, ((0, CP - 21), (0, 0)))

    T0p, s0t, d0t = _tc_prep(ftp, W0, att_s0.reshape(D, 1),
                             att_d0.reshape(D, 1))

    cflat = _get_sc_layer0()(xp, pk, s0t.reshape(CP), d0t.reshape(CP))
    cmat = cflat.reshape(NP, CP)

    H1, as1, ad1 = _tc_mid(cmat, T0p, b0.reshape(1, D), W1,
                           att_s1.reshape(D, 1), att_d1.reshape(D, 1))

    o1 = _get_sc_layer1()(H1, as1.reshape(NP), ad1.reshape(NP), pk)

    batchp = jnp.pad(batch.astype(_i32), (0, NP - n),
                     constant_values=B).reshape(NP, 1)
    return _tc_final(o1, b1.reshape(1, D), batchp, fW1,
                     fb1.reshape(1, 2 * D), fW2, fb2.reshape(1, T))
